# Initial kernel scaffold; baseline (speedup 1.0000x reference)
import jax
import jax.numpy as jnp
from jax.experimental import pallas as pl


def kernel(af, bf, ff, W_e, b_e, W_a, b_a, W_n, b_n, Wf_e, bf_e, Wf_a, bf_a, Wf_n, bf_n, a2a_edge_index, a2f_src, a2f_dst, f2f_edge_index):
    raise NotImplementedError("write your pallas kernel here")


# speedup vs baseline: 4.9587x; 4.9587x over previous
import functools

import jax
import jax.numpy as jnp
from jax import lax
from jax.experimental import pallas as pl
from jax.experimental.pallas import tpu as pltpu
from jax.experimental.pallas import tpu_sc as plsc

N_ATOM = 10000
E_A2A = 320000
N_FG = 2000
E_A2F = 10000
E_F2F = 16000
D = 128
DE = 16
H = 128
UFF = H + D

NC = 2
NS = 16
NW = NC * NS


def _mesh():
    return plsc.VectorSubcoreMesh(core_axis_name="c", subcore_axis_name="s")


def _zero_acc(zeros_hbm, acc_sh, s, n_rows):
    nblk = n_rows // 200
    full, extra = nblk // NS, nblk % NS
    nit = jnp.where(s < extra, full + 1, full)

    def zb(k, _):
        pltpu.sync_copy(zeros_hbm, acc_sh.at[pl.ds((k * NS + s) * 200, 200)])
        return 0

    lax.fori_loop(0, nit, zb, 0)


def _scale_rows(ue_v, ee_v, ch):
    def grp(gi, _):
        v = ee_v[pl.ds(gi * 16, 16)]
        for e in range(16):
            ev = v[e]
            r = gi * 16 + e
            for cg in range(H // 16):
                sl = pl.ds(cg * 16, 16)
                ue_v[r, sl] = ue_v[r, sl] * ev
        return 0

    lax.fori_loop(0, ch // 16, grp, 0)


def _proj_node_body(af_ref, ws_ref, wd_ref, be_ref, ps_ref, pd_ref):
    a = af_ref[...]
    ps_ref[...] = jnp.dot(a, ws_ref[...], preferred_element_type=jnp.float32)
    pd_ref[...] = (jnp.dot(a, wd_ref[...], preferred_element_type=jnp.float32)
                   + be_ref[...])


def _proj_node(af, ws, wd, be2d, n_rows, blk):
    grid = (n_rows // blk,)
    return pl.pallas_call(
        _proj_node_body,
        grid=grid,
        in_specs=[
            pl.BlockSpec((blk, af.shape[1]), lambda i: (i, 0)),
            pl.BlockSpec(ws.shape, lambda i: (0, 0)),
            pl.BlockSpec(wd.shape, lambda i: (0, 0)),
            pl.BlockSpec(be2d.shape, lambda i: (0, 0)),
        ],
        out_specs=[
            pl.BlockSpec((blk, H), lambda i: (i, 0)),
            pl.BlockSpec((blk, H), lambda i: (i, 0)),
        ],
        out_shape=[
            jax.ShapeDtypeStruct((n_rows, H), jnp.float32),
            jax.ShapeDtypeStruct((n_rows, H), jnp.float32),
        ],
    )(af, ws, wd, be2d)


def _sc_gather_add_body(epw, ch, ps_hbm, pd_hbm, src_hbm, dst_hbm, g_hbm,
                        si_v, di_v, gs_v, gd_v, sem1, sem2):
    c = lax.axis_index("c")
    s = lax.axis_index("s")
    wid = c * NS + s
    nit = epw // ch

    def body(i, _):
        base = wid * epw + i * ch
        pltpu.sync_copy(src_hbm.at[pl.ds(base, ch)], si_v)
        pltpu.sync_copy(dst_hbm.at[pl.ds(base, ch)], di_v)
        cp1 = pltpu.async_copy(ps_hbm.at[si_v], gs_v, sem1)
        cp2 = pltpu.async_copy(pd_hbm.at[di_v], gd_v, sem2)
        cp1.wait()
        cp2.wait()

        def add_row(r, _):
            for cg in range(H // 16):
                sl = pl.ds(cg * 16, 16)
                gs_v[r, sl] = gs_v[r, sl] + gd_v[r, sl]
            return 0

        lax.fori_loop(0, ch, add_row, 0)
        pltpu.sync_copy(gs_v, g_hbm.at[pl.ds(base, ch)])
        return 0

    lax.fori_loop(0, nit, body, 0)


def _sc_gather_add(ps, pd, src, dst, n_edges, ch):
    epw = n_edges // NW
    body = functools.partial(_sc_gather_add_body, epw, ch)
    k = pl.kernel(
        body,
        mesh=_mesh(),
        out_type=jax.ShapeDtypeStruct((n_edges, H), jnp.float32),
        scratch_types=[
            pltpu.VMEM((ch,), jnp.int32),
            pltpu.VMEM((ch,), jnp.int32),
            pltpu.VMEM((ch, H), jnp.float32),
            pltpu.VMEM((ch, H), jnp.float32),
            pltpu.SemaphoreType.DMA,
            pltpu.SemaphoreType.DMA,
        ],
    )
    return k(ps, pd, src, dst)


def _edge_act_body(g_ref, bfe_ref, we_ref, wa_ref, ba_ref, ue_ref, ee_ref):
    pre = g_ref[...] + jnp.dot(bfe_ref[...], we_ref[...],
                               preferred_element_type=jnp.float32)
    ue = jnp.maximum(pre, 0.0)
    ue_ref[...] = ue
    logit = jnp.dot(ue, wa_ref[...], preferred_element_type=jnp.float32)
    ee_ref[...] = jnp.exp(logit + ba_ref[...])


def _edge_act(g, bf, we_e, wa, ba2d, blk):
    n = g.shape[0]
    grid = (n // blk,)
    return pl.pallas_call(
        _edge_act_body,
        grid=grid,
        in_specs=[
            pl.BlockSpec((blk, H), lambda i: (i, 0)),
            pl.BlockSpec((blk, DE), lambda i: (i, 0)),
            pl.BlockSpec(we_e.shape, lambda i: (0, 0)),
            pl.BlockSpec(wa.shape, lambda i: (0, 0)),
            pl.BlockSpec(ba2d.shape, lambda i: (0, 0)),
        ],
        out_specs=[
            pl.BlockSpec((blk, H), lambda i: (i, 0)),
            pl.BlockSpec((blk, 1), lambda i: (i, 0)),
        ],
        out_shape=[
            jax.ShapeDtypeStruct((n, H), jnp.float32),
            jax.ShapeDtypeStruct((n, 1), jnp.float32),
        ],
    )(g, bf, we_e, wa, ba2d)


def _sc_scatter_body(n_nodes, ch, base_of, nit_of, ue_hbm, ee_hbm, dst_hbm,
                     z_hbm, nump_hbm, ue_v, ee_v, di_v, acc_sh, sem):
    c = lax.axis_index("c")
    s = lax.axis_index("s")
    wid = c * NS + s
    _zero_acc(z_hbm, acc_sh, s, n_nodes)
    plsc.subcore_barrier()

    def body(i, _):
        base = base_of(wid, i)
        pltpu.sync_copy(ue_hbm.at[pl.ds(base, ch)], ue_v)
        pltpu.sync_copy(ee_hbm.at[pl.ds(base, ch)], ee_v)
        pltpu.sync_copy(dst_hbm.at[pl.ds(base, ch)], di_v)
        _scale_rows(ue_v, ee_v, ch)
        pltpu.sync_copy(ue_v, acc_sh.at[di_v], add=True)
        return 0

    lax.fori_loop(0, nit_of(wid), body, 0)
    plsc.subcore_barrier()

    @pl.when(s == 0)
    def _():
        pltpu.sync_copy(acc_sh, nump_hbm.at[c])


def _sc_scatter(ue, ee, dst, zeros, n_nodes, ch, base_of, nit_of):
    body = functools.partial(_sc_scatter_body, n_nodes, ch, base_of, nit_of)
    k = pl.kernel(
        body,
        mesh=_mesh(),
        out_type=jax.ShapeDtypeStruct((NC, n_nodes, H), jnp.float32),
        scratch_types=[
            pltpu.VMEM((ch, H), jnp.float32),
            pltpu.VMEM((ch,), jnp.float32),
            pltpu.VMEM((ch,), jnp.int32),
            pltpu.VMEM_SHARED((n_nodes, H), jnp.float32),
            pltpu.SemaphoreType.DMA,
        ],
    )
    return k(ue, ee, dst, zeros)


def _sc_sdenom_body(n_nodes, ch, base_of, nit_of, ee_hbm, dst_hbm, sp_hbm,
                    ee_v, di_v, sacc_v):
    c = lax.axis_index("c")
    s = lax.axis_index("s")
    wid = c * NS + s

    def zs(i, _):
        sacc_v[pl.ds(i * 16, 16)] = jnp.zeros((16,), jnp.float32)
        return 0

    lax.fori_loop(0, n_nodes // 16, zs, 0)

    def body(i, _):
        base = base_of(wid, i)
        pltpu.sync_copy(ee_hbm.at[pl.ds(base, ch)], ee_v)
        pltpu.sync_copy(dst_hbm.at[pl.ds(base, ch)], di_v)
        for g in range(ch // 16):
            sl = pl.ds(g * 16, 16)
            plsc.addupdate_scatter(sacc_v, [di_v[sl]], ee_v[sl])
        return 0

    lax.fori_loop(0, nit_of(wid), body, 0)
    pltpu.sync_copy(sacc_v, sp_hbm.at[wid])


def _sc_sdenom(ee, dst, n_nodes, ch, base_of, nit_of):
    body = functools.partial(_sc_sdenom_body, n_nodes, ch, base_of, nit_of)
    k = pl.kernel(
        body,
        mesh=_mesh(),
        out_type=jax.ShapeDtypeStruct((NW, n_nodes), jnp.float32),
        scratch_types=[
            pltpu.VMEM((ch,), jnp.float32),
            pltpu.VMEM((ch,), jnp.int32),
            pltpu.VMEM((n_nodes,), jnp.float32),
        ],
        compiler_params=pltpu.CompilerParams(needs_layout_passes=False),
    )
    return k(ee, dst)


def _node_upd_body(np_ref, sp_ref, x_ref, wx_ref, wa_ref, b_ref, out_ref):
    num = np_ref[0] + np_ref[1]
    s = jnp.sum(sp_ref[...], axis=0)
    agg = num / (s[:, None] + 1e-9)
    out_ref[...] = jnp.maximum(
        jnp.dot(x_ref[...], wx_ref[...], preferred_element_type=jnp.float32)
        + jnp.dot(agg, wa_ref[...], preferred_element_type=jnp.float32)
        + b_ref[...], 0.0)


def _node_upd(nump, sp, x, wx, wa, b2d):
    n = x.shape[0]
    return pl.pallas_call(
        _node_upd_body,
        out_shape=jax.ShapeDtypeStruct((n, H), jnp.float32),
    )(nump, sp, x, wx, wa, b2d)


def _sc_a2f_body(uaf_hbm, src_hbm, dst_hbm, z_hbm, aggp_hbm,
                 si_v, di_v, rows_v, acc_sh, sem):
    c = lax.axis_index("c")
    s = lax.axis_index("s")
    wid = c * NS + s
    _zero_acc(z_hbm, acc_sh, s, N_FG)
    plsc.subcore_barrier()

    nit = jnp.where(wid < NW - 1, 4, 1)

    def body(i, _):
        base = wid * 320 + i * 80
        pltpu.sync_copy(src_hbm.at[pl.ds(base, 80)], si_v)
        pltpu.sync_copy(dst_hbm.at[pl.ds(base, 80)], di_v)
        pltpu.async_copy(uaf_hbm.at[si_v], rows_v, sem).wait()
        pltpu.sync_copy(rows_v, acc_sh.at[di_v], add=True)
        return 0

    lax.fori_loop(0, nit, body, 0)
    plsc.subcore_barrier()

    @pl.when(s == 0)
    def _():
        pltpu.sync_copy(acc_sh, aggp_hbm.at[c])


def _sc_a2f(uaf, src, dst, zeros):
    k = pl.kernel(
        _sc_a2f_body,
        mesh=_mesh(),
        out_type=jax.ShapeDtypeStruct((NC, N_FG, H), jnp.float32),
        scratch_types=[
            pltpu.VMEM((80,), jnp.int32),
            pltpu.VMEM((80,), jnp.int32),
            pltpu.VMEM((80, H), jnp.float32),
            pltpu.VMEM_SHARED((N_FG, H), jnp.float32),
            pltpu.SemaphoreType.DMA,
        ],
    )
    return k(uaf, src, dst, zeros)


def _fg_proj_body(aggp_ref, ff_ref, wf_ref, bfe_ref, uff_ref, pfs_ref, pfd_ref):
    aggm = aggp_ref[0] + aggp_ref[1]
    uff = jnp.concatenate([aggm, ff_ref[...]], axis=-1)
    uff_ref[...] = uff
    pfs_ref[...] = jnp.dot(uff, wf_ref[0:UFF, :],
                           preferred_element_type=jnp.float32)
    pfd_ref[...] = (jnp.dot(uff, wf_ref[UFF:2 * UFF, :],
                            preferred_element_type=jnp.float32)
                    + bfe_ref[...])


def _fg_proj(aggp, ff, wf_e, bfe2d):
    return pl.pallas_call(
        _fg_proj_body,
        out_shape=[
            jax.ShapeDtypeStruct((N_FG, UFF), jnp.float32),
            jax.ShapeDtypeStruct((N_FG, H), jnp.float32),
            jax.ShapeDtypeStruct((N_FG, H), jnp.float32),
        ],
    )(aggp, ff, wf_e, bfe2d)


def _sc_f2f_gather_body(ps_hbm, pd_hbm, src_hbm, dst_hbm, g_hbm,
                        si_v, di_v, gs_v, gd_v, sem1, sem2):
    c = lax.axis_index("c")
    s = lax.axis_index("s")
    wid = c * NS + s
    nit = jnp.where(wid < NW - 1, 8, 2)

    def body(i, _):
        base = wid * 512 + i * 64
        pltpu.sync_copy(src_hbm.at[pl.ds(base, 64)], si_v)
        pltpu.sync_copy(dst_hbm.at[pl.ds(base, 64)], di_v)
        cp1 = pltpu.async_copy(ps_hbm.at[si_v], gs_v, sem1)
        cp2 = pltpu.async_copy(pd_hbm.at[di_v], gd_v, sem2)
        cp1.wait()
        cp2.wait()

        def add_row(r, _):
            for cg in range(H // 16):
                sl = pl.ds(cg * 16, 16)
                gs_v[r, sl] = gs_v[r, sl] + gd_v[r, sl]
            return 0

        lax.fori_loop(0, 64, add_row, 0)
        pltpu.sync_copy(gs_v, g_hbm.at[pl.ds(base, 64)])
        return 0

    lax.fori_loop(0, nit, body, 0)


def _sc_f2f_gather(pfs, pfd, src, dst):
    k = pl.kernel(
        _sc_f2f_gather_body,
        mesh=_mesh(),
        out_type=jax.ShapeDtypeStruct((E_F2F, H), jnp.float32),
        scratch_types=[
            pltpu.VMEM((64,), jnp.int32),
            pltpu.VMEM((64,), jnp.int32),
            pltpu.VMEM((64, H), jnp.float32),
            pltpu.VMEM((64, H), jnp.float32),
            pltpu.SemaphoreType.DMA,
            pltpu.SemaphoreType.DMA,
        ],
    )
    return k(pfs, pfd, src, dst)


def _f2f_act_body(g_ref, wa_ref, ba_ref, ue_ref, ee_ref):
    ue = jnp.maximum(g_ref[...], 0.0)
    ue_ref[...] = ue
    logit = jnp.dot(ue, wa_ref[...], preferred_element_type=jnp.float32)
    ee_ref[...] = jnp.exp(logit + ba_ref[...])


def _f2f_act(g, wa, ba2d, blk):
    n = g.shape[0]
    return pl.pallas_call(
        _f2f_act_body,
        grid=(n // blk,),
        in_specs=[
            pl.BlockSpec((blk, H), lambda i: (i, 0)),
            pl.BlockSpec(wa.shape, lambda i: (0, 0)),
            pl.BlockSpec(ba2d.shape, lambda i: (0, 0)),
        ],
        out_specs=[
            pl.BlockSpec((blk, H), lambda i: (i, 0)),
            pl.BlockSpec((blk, 1), lambda i: (i, 0)),
        ],
        out_shape=[
            jax.ShapeDtypeStruct((n, H), jnp.float32),
            jax.ShapeDtypeStruct((n, 1), jnp.float32),
        ],
    )(g, wa, ba2d)


def _fg_upd_body(np_ref, sp_ref, uff_ref, wf_ref, b_ref, out_ref):
    num = np_ref[0] + np_ref[1]
    s = jnp.sum(sp_ref[...], axis=0)
    agg = num / (s[:, None] + 1e-9)
    uff = uff_ref[...]
    out_ref[...] = jnp.maximum(
        jnp.dot(uff, wf_ref[0:UFF, :], preferred_element_type=jnp.float32)
        + jnp.dot(agg, wf_ref[UFF:UFF + H, :],
                  preferred_element_type=jnp.float32)
        + b_ref[...], 0.0)


def _fg_upd(nump, sp, uff, wf_n, b2d):
    return pl.pallas_call(
        _fg_upd_body,
        out_shape=jax.ShapeDtypeStruct((N_FG, H), jnp.float32),
    )(nump, sp, uff, wf_n, b2d)


def kernel(af, bf, ff, W_e, b_e, W_a, b_a, W_n, b_n, Wf_e, bf_e, Wf_a, bf_a,
           Wf_n, bf_n, a2a_edge_index, a2f_src, a2f_dst, f2f_edge_index):
    src = a2a_edge_index[0].astype(jnp.int32)
    dst = a2a_edge_index[1].astype(jnp.int32)
    fsrc = f2f_edge_index[0].astype(jnp.int32)
    fdst = f2f_edge_index[1].astype(jnp.int32)
    asrc = a2f_src.astype(jnp.int32)
    adst = a2f_dst.astype(jnp.int32)

    ps, pd = _proj_node(af, W_e[:D], W_e[D:2 * D], b_e.reshape(1, H),
                        N_ATOM, 1000)
    g = _sc_gather_add(ps, pd, src, dst, E_A2A, 80)
    ubf, ee = _edge_act(g, bf, W_e[2 * D:], W_a, b_a.reshape(1, 1), 2000)
    ee = ee.reshape(E_A2A)
    zeros = jnp.zeros((200, H), jnp.float32)
    epw = E_A2A // NW
    a2a_base = lambda wid, i: wid * epw + i * 80
    a2a_nit = lambda wid: epw // 80
    nump = _sc_scatter(ubf, ee, dst, zeros, N_ATOM, 80, a2a_base, a2a_nit)
    sp = _sc_sdenom(ee, dst, N_ATOM, 80, a2a_base, a2a_nit)
    uaf = _node_upd(nump, sp, af, W_n[:D], W_n[D:], b_n.reshape(1, H))
    aggp = _sc_a2f(uaf, asrc, adst, zeros)
    uff, pfs, pfd = _fg_proj(aggp, ff, Wf_e, bf_e.reshape(1, H))
    gf = _sc_f2f_gather(pfs, pfd, fsrc, fdst)
    uef, eef = _f2f_act(gf, Wf_a, bf_a.reshape(1, 1), 2000)
    eef = eef.reshape(E_F2F)
    f2f_base = lambda wid, i: wid * 512 + i * 64
    f2f_nit = lambda wid: jnp.where(wid < NW - 1, 8, 2)
    numfp = _sc_scatter(uef, eef, fdst, zeros, N_FG, 64, f2f_base, f2f_nit)
    sfp = _sc_sdenom(eef, fdst, N_FG, 64, f2f_base, f2f_nit)
    conv_uff = _fg_upd(numfp, sfp, uff, Wf_n, bf_n.reshape(1, H))
    return (uaf, ubf, conv_uff)


# speedup vs baseline: 5.7261x; 1.1547x over previous
import functools

import jax
import jax.numpy as jnp
from jax import lax
from jax.experimental import pallas as pl
from jax.experimental.pallas import tpu as pltpu
from jax.experimental.pallas import tpu_sc as plsc

N_ATOM = 10000
E_A2A = 320000
N_FG = 2000
E_A2F = 10000
E_F2F = 16000
D = 128
DE = 16
H = 128
UFF = H + D

NC = 2
NS = 16
NW = NC * NS


def _mesh():
    return plsc.VectorSubcoreMesh(core_axis_name="c", subcore_axis_name="s")


def _zero_acc(zeros_hbm, acc_sh, s, n_rows):
    nblk = n_rows // 200
    full, extra = nblk // NS, nblk % NS
    nit = jnp.where(s < extra, full + 1, full)

    def zb(k, _):
        pltpu.sync_copy(zeros_hbm, acc_sh.at[pl.ds((k * NS + s) * 200, 200)])
        return 0

    lax.fori_loop(0, nit, zb, 0)


def _proj_node_body(af_ref, ws_ref, wd_ref, be_ref, ps_ref, pd_ref):
    a = af_ref[...]
    ps_ref[...] = jnp.dot(a, ws_ref[...], preferred_element_type=jnp.float32)
    pd_ref[...] = (jnp.dot(a, wd_ref[...], preferred_element_type=jnp.float32)
                   + be_ref[...])


def _proj_node(af, ws, wd, be2d, n_rows, blk):
    grid = (n_rows // blk,)
    return pl.pallas_call(
        _proj_node_body,
        grid=grid,
        in_specs=[
            pl.BlockSpec((blk, af.shape[1]), lambda i: (i, 0)),
            pl.BlockSpec(ws.shape, lambda i: (0, 0)),
            pl.BlockSpec(wd.shape, lambda i: (0, 0)),
            pl.BlockSpec(be2d.shape, lambda i: (0, 0)),
        ],
        out_specs=[
            pl.BlockSpec((blk, H), lambda i: (i, 0)),
            pl.BlockSpec((blk, H), lambda i: (i, 0)),
        ],
        out_shape=[
            jax.ShapeDtypeStruct((n_rows, H), jnp.float32),
            jax.ShapeDtypeStruct((n_rows, H), jnp.float32),
        ],
    )(af, ws, wd, be2d)


def _sc_gather_add_body(epw, sub, nsub, ps_hbm, pd_hbm, src_hbm, dst_hbm,
                        g_hbm, si_v, di_v, gs_v, gd_v, sem1, sem2):
    c = lax.axis_index("c")
    s = lax.axis_index("s")
    wid = c * NS + s
    ch = sub * nsub
    nit = epw // ch

    def body(i, _):
        base = wid * epw + i * ch
        pltpu.sync_copy(src_hbm.at[pl.ds(base, ch)], si_v)
        pltpu.sync_copy(dst_hbm.at[pl.ds(base, ch)], di_v)
        for j in range(nsub):
            sl = pl.ds(j * sub, sub)
            cp1 = pltpu.async_copy(ps_hbm.at[si_v.at[sl]], gs_v.at[sl], sem1)
            cp2 = pltpu.async_copy(pd_hbm.at[di_v.at[sl]], gd_v.at[sl], sem2)
            cp1.wait()
            cp2.wait()

        def add_row(r, _):
            for cg in range(H // 16):
                sl = pl.ds(cg * 16, 16)
                gs_v[r, sl] = gs_v[r, sl] + gd_v[r, sl]
            return 0

        lax.fori_loop(0, ch, add_row, 0)
        pltpu.sync_copy(gs_v, g_hbm.at[pl.ds(base, ch)])
        return 0

    lax.fori_loop(0, nit, body, 0)


def _sc_gather_add(ps, pd, src, dst, n_edges, sub, nsub):
    epw = n_edges // NW
    ch = sub * nsub
    body = functools.partial(_sc_gather_add_body, epw, sub, nsub)
    k = pl.kernel(
        body,
        mesh=_mesh(),
        out_type=jax.ShapeDtypeStruct((n_edges, H), jnp.float32),
        scratch_types=[
            pltpu.VMEM((ch,), jnp.int32),
            pltpu.VMEM((ch,), jnp.int32),
            pltpu.VMEM((ch, H), jnp.float32),
            pltpu.VMEM((ch, H), jnp.float32),
            pltpu.SemaphoreType.DMA,
            pltpu.SemaphoreType.DMA,
        ],
    )
    return k(ps, pd, src, dst)


def _edge_act_body(g_ref, bfe_ref, we_ref, wa_ref, ba_ref, ue_ref, ee_ref,
                   wue_ref):
    pre = g_ref[...] + jnp.dot(bfe_ref[...], we_ref[...],
                               preferred_element_type=jnp.float32)
    ue = jnp.maximum(pre, 0.0)
    ue_ref[...] = ue
    logit = jnp.dot(ue, wa_ref[...], preferred_element_type=jnp.float32)
    ee = jnp.exp(logit + ba_ref[...])
    ee_ref[...] = ee
    wue_ref[...] = ue * ee


def _edge_act(g, bf, we_e, wa, ba2d, blk):
    n = g.shape[0]
    grid = (n // blk,)
    return pl.pallas_call(
        _edge_act_body,
        grid=grid,
        in_specs=[
            pl.BlockSpec((blk, H), lambda i: (i, 0)),
            pl.BlockSpec((blk, DE), lambda i: (i, 0)),
            pl.BlockSpec(we_e.shape, lambda i: (0, 0)),
            pl.BlockSpec(wa.shape, lambda i: (0, 0)),
            pl.BlockSpec(ba2d.shape, lambda i: (0, 0)),
        ],
        out_specs=[
            pl.BlockSpec((blk, H), lambda i: (i, 0)),
            pl.BlockSpec((blk, 1), lambda i: (i, 0)),
            pl.BlockSpec((blk, H), lambda i: (i, 0)),
        ],
        out_shape=[
            jax.ShapeDtypeStruct((n, H), jnp.float32),
            jax.ShapeDtypeStruct((n, 1), jnp.float32),
            jax.ShapeDtypeStruct((n, H), jnp.float32),
        ],
    )(g, bf, we_e, wa, ba2d)


def _sc_scatter_body(n_nodes, ch, base_of, nit_of, wue_hbm, dst_hbm,
                     z_hbm, nump_hbm, w_v, di_v, acc_sh, sem):
    c = lax.axis_index("c")
    s = lax.axis_index("s")
    wid = c * NS + s
    _zero_acc(z_hbm, acc_sh, s, n_nodes)
    plsc.subcore_barrier()

    def body(i, _):
        base = base_of(wid, i)
        pltpu.sync_copy(wue_hbm.at[pl.ds(base, ch)], w_v)
        pltpu.sync_copy(dst_hbm.at[pl.ds(base, ch)], di_v)
        pltpu.sync_copy(w_v, acc_sh.at[di_v], add=True)
        return 0

    lax.fori_loop(0, nit_of(wid), body, 0)
    plsc.subcore_barrier()

    @pl.when(s == 0)
    def _():
        pltpu.sync_copy(acc_sh, nump_hbm.at[c])


def _sc_scatter(wue, dst, zeros, n_nodes, ch, base_of, nit_of):
    body = functools.partial(_sc_scatter_body, n_nodes, ch, base_of, nit_of)
    k = pl.kernel(
        body,
        mesh=_mesh(),
        out_type=jax.ShapeDtypeStruct((NC, n_nodes, H), jnp.float32),
        scratch_types=[
            pltpu.VMEM((ch, H), jnp.float32),
            pltpu.VMEM((ch,), jnp.int32),
            pltpu.VMEM_SHARED((n_nodes, H), jnp.float32),
            pltpu.SemaphoreType.DMA,
        ],
    )
    return k(wue, dst, zeros)


def _sc_sdenom_body(n_nodes, ch, base_of, nit_of, ee_hbm, dst_hbm, sp_hbm,
                    ee_v, di_v, sacc_v):
    c = lax.axis_index("c")
    s = lax.axis_index("s")
    wid = c * NS + s

    def zs(i, _):
        sacc_v[pl.ds(i * 16, 16)] = jnp.zeros((16,), jnp.float32)
        return 0

    lax.fori_loop(0, n_nodes // 16, zs, 0)

    def body(i, _):
        base = base_of(wid, i)
        pltpu.sync_copy(ee_hbm.at[pl.ds(base, ch)], ee_v)
        pltpu.sync_copy(dst_hbm.at[pl.ds(base, ch)], di_v)
        for g in range(ch // 16):
            sl = pl.ds(g * 16, 16)
            plsc.addupdate_scatter(sacc_v, [di_v[sl]], ee_v[sl])
        return 0

    lax.fori_loop(0, nit_of(wid), body, 0)
    pltpu.sync_copy(sacc_v, sp_hbm.at[wid])


def _sc_sdenom(ee, dst, n_nodes, ch, base_of, nit_of):
    body = functools.partial(_sc_sdenom_body, n_nodes, ch, base_of, nit_of)
    k = pl.kernel(
        body,
        mesh=_mesh(),
        out_type=jax.ShapeDtypeStruct((NW, n_nodes), jnp.float32),
        scratch_types=[
            pltpu.VMEM((ch,), jnp.float32),
            pltpu.VMEM((ch,), jnp.int32),
            pltpu.VMEM((n_nodes,), jnp.float32),
        ],
        compiler_params=pltpu.CompilerParams(needs_layout_passes=False),
    )
    return k(ee, dst)


def _node_upd_body(np_ref, sp_ref, x_ref, wx_ref, wa_ref, b_ref, out_ref):
    num = np_ref[0] + np_ref[1]
    s = jnp.sum(sp_ref[...], axis=0)
    agg = num / (s[:, None] + 1e-9)
    out_ref[...] = jnp.maximum(
        jnp.dot(x_ref[...], wx_ref[...], preferred_element_type=jnp.float32)
        + jnp.dot(agg, wa_ref[...], preferred_element_type=jnp.float32)
        + b_ref[...], 0.0)


def _node_upd(nump, sp, x, wx, wa, b2d):
    n = x.shape[0]
    return pl.pallas_call(
        _node_upd_body,
        out_shape=jax.ShapeDtypeStruct((n, H), jnp.float32),
    )(nump, sp, x, wx, wa, b2d)


def _sc_a2f_body(uaf_hbm, src_hbm, dst_hbm, z_hbm, aggp_hbm,
                 si_v, di_v, rows_v, acc_sh, sem):
    c = lax.axis_index("c")
    s = lax.axis_index("s")
    wid = c * NS + s
    _zero_acc(z_hbm, acc_sh, s, N_FG)
    plsc.subcore_barrier()

    nit = jnp.where(wid < NW - 1, 4, 1)

    def body(i, _):
        base = wid * 320 + i * 80
        pltpu.sync_copy(src_hbm.at[pl.ds(base, 80)], si_v)
        pltpu.sync_copy(dst_hbm.at[pl.ds(base, 80)], di_v)
        pltpu.async_copy(uaf_hbm.at[si_v], rows_v, sem).wait()
        pltpu.sync_copy(rows_v, acc_sh.at[di_v], add=True)
        return 0

    lax.fori_loop(0, nit, body, 0)
    plsc.subcore_barrier()

    @pl.when(s == 0)
    def _():
        pltpu.sync_copy(acc_sh, aggp_hbm.at[c])


def _sc_a2f(uaf, src, dst, zeros):
    k = pl.kernel(
        _sc_a2f_body,
        mesh=_mesh(),
        out_type=jax.ShapeDtypeStruct((NC, N_FG, H), jnp.float32),
        scratch_types=[
            pltpu.VMEM((80,), jnp.int32),
            pltpu.VMEM((80,), jnp.int32),
            pltpu.VMEM((80, H), jnp.float32),
            pltpu.VMEM_SHARED((N_FG, H), jnp.float32),
            pltpu.SemaphoreType.DMA,
        ],
    )
    return k(uaf, src, dst, zeros)


def _fg_proj_body(aggp_ref, ff_ref, wf_ref, bfe_ref, uff_ref, pfs_ref, pfd_ref):
    aggm = aggp_ref[0] + aggp_ref[1]
    uff = jnp.concatenate([aggm, ff_ref[...]], axis=-1)
    uff_ref[...] = uff
    pfs_ref[...] = jnp.dot(uff, wf_ref[0:UFF, :],
                           preferred_element_type=jnp.float32)
    pfd_ref[...] = (jnp.dot(uff, wf_ref[UFF:2 * UFF, :],
                            preferred_element_type=jnp.float32)
                    + bfe_ref[...])


def _fg_proj(aggp, ff, wf_e, bfe2d):
    return pl.pallas_call(
        _fg_proj_body,
        out_shape=[
            jax.ShapeDtypeStruct((N_FG, UFF), jnp.float32),
            jax.ShapeDtypeStruct((N_FG, H), jnp.float32),
            jax.ShapeDtypeStruct((N_FG, H), jnp.float32),
        ],
    )(aggp, ff, wf_e, bfe2d)


def _sc_f2f_gather_body(ps_hbm, pd_hbm, src_hbm, dst_hbm, g_hbm,
                        si_v, di_v, gs_v, gd_v, sem1, sem2):
    c = lax.axis_index("c")
    s = lax.axis_index("s")
    wid = c * NS + s
    nit = jnp.where(wid < NW - 1, 8, 2)

    def body(i, _):
        base = wid * 512 + i * 64
        pltpu.sync_copy(src_hbm.at[pl.ds(base, 64)], si_v)
        pltpu.sync_copy(dst_hbm.at[pl.ds(base, 64)], di_v)
        cp1 = pltpu.async_copy(ps_hbm.at[si_v], gs_v, sem1)
        cp2 = pltpu.async_copy(pd_hbm.at[di_v], gd_v, sem2)
        cp1.wait()
        cp2.wait()

        def add_row(r, _):
            for cg in range(H // 16):
                sl = pl.ds(cg * 16, 16)
                gs_v[r, sl] = gs_v[r, sl] + gd_v[r, sl]
            return 0

        lax.fori_loop(0, 64, add_row, 0)
        pltpu.sync_copy(gs_v, g_hbm.at[pl.ds(base, 64)])
        return 0

    lax.fori_loop(0, nit, body, 0)


def _sc_f2f_gather(pfs, pfd, src, dst):
    k = pl.kernel(
        _sc_f2f_gather_body,
        mesh=_mesh(),
        out_type=jax.ShapeDtypeStruct((E_F2F, H), jnp.float32),
        scratch_types=[
            pltpu.VMEM((64,), jnp.int32),
            pltpu.VMEM((64,), jnp.int32),
            pltpu.VMEM((64, H), jnp.float32),
            pltpu.VMEM((64, H), jnp.float32),
            pltpu.SemaphoreType.DMA,
            pltpu.SemaphoreType.DMA,
        ],
    )
    return k(pfs, pfd, src, dst)


def _f2f_act_body(g_ref, wa_ref, ba_ref, ue_ref, ee_ref, wue_ref):
    ue = jnp.maximum(g_ref[...], 0.0)
    ue_ref[...] = ue
    logit = jnp.dot(ue, wa_ref[...], preferred_element_type=jnp.float32)
    ee = jnp.exp(logit + ba_ref[...])
    ee_ref[...] = ee
    wue_ref[...] = ue * ee


def _f2f_act(g, wa, ba2d, blk):
    n = g.shape[0]
    return pl.pallas_call(
        _f2f_act_body,
        grid=(n // blk,),
        in_specs=[
            pl.BlockSpec((blk, H), lambda i: (i, 0)),
            pl.BlockSpec(wa.shape, lambda i: (0, 0)),
            pl.BlockSpec(ba2d.shape, lambda i: (0, 0)),
        ],
        out_specs=[
            pl.BlockSpec((blk, H), lambda i: (i, 0)),
            pl.BlockSpec((blk, 1), lambda i: (i, 0)),
            pl.BlockSpec((blk, H), lambda i: (i, 0)),
        ],
        out_shape=[
            jax.ShapeDtypeStruct((n, H), jnp.float32),
            jax.ShapeDtypeStruct((n, 1), jnp.float32),
            jax.ShapeDtypeStruct((n, H), jnp.float32),
        ],
    )(g, wa, ba2d)


def _fg_upd_body(np_ref, sp_ref, uff_ref, wf_ref, b_ref, out_ref):
    num = np_ref[0] + np_ref[1]
    s = jnp.sum(sp_ref[...], axis=0)
    agg = num / (s[:, None] + 1e-9)
    uff = uff_ref[...]
    out_ref[...] = jnp.maximum(
        jnp.dot(uff, wf_ref[0:UFF, :], preferred_element_type=jnp.float32)
        + jnp.dot(agg, wf_ref[UFF:UFF + H, :],
                  preferred_element_type=jnp.float32)
        + b_ref[...], 0.0)


def _fg_upd(nump, sp, uff, wf_n, b2d):
    return pl.pallas_call(
        _fg_upd_body,
        out_shape=jax.ShapeDtypeStruct((N_FG, H), jnp.float32),
    )(nump, sp, uff, wf_n, b2d)


def kernel(af, bf, ff, W_e, b_e, W_a, b_a, W_n, b_n, Wf_e, bf_e, Wf_a, bf_a,
           Wf_n, bf_n, a2a_edge_index, a2f_src, a2f_dst, f2f_edge_index):
    src = a2a_edge_index[0].astype(jnp.int32)
    dst = a2a_edge_index[1].astype(jnp.int32)
    fsrc = f2f_edge_index[0].astype(jnp.int32)
    fdst = f2f_edge_index[1].astype(jnp.int32)
    asrc = a2f_src.astype(jnp.int32)
    adst = a2f_dst.astype(jnp.int32)

    ps, pd = _proj_node(af, W_e[:D], W_e[D:2 * D], b_e.reshape(1, H),
                        N_ATOM, 1000)
    g = _sc_gather_add(ps, pd, src, dst, E_A2A, 80, 5)
    ubf, ee, wue = _edge_act(g, bf, W_e[2 * D:], W_a, b_a.reshape(1, 1), 2000)
    ee = ee.reshape(E_A2A)
    zeros = jnp.zeros((200, H), jnp.float32)
    epw = E_A2A // NW
    nump = _sc_scatter(wue, dst, zeros, N_ATOM, 80,
                       lambda wid, i: wid * epw + i * 80,
                       lambda wid: epw // 80)
    sp = _sc_sdenom(ee, dst, N_ATOM, 80,
                    lambda wid, i: wid * epw + i * 80,
                    lambda wid: epw // 80)
    uaf = _node_upd(nump, sp, af, W_n[:D], W_n[D:], b_n.reshape(1, H))
    aggp = _sc_a2f(uaf, asrc, adst, zeros)
    uff, pfs, pfd = _fg_proj(aggp, ff, Wf_e, bf_e.reshape(1, H))
    gf = _sc_f2f_gather(pfs, pfd, fsrc, fdst)
    uef, eef, wuef = _f2f_act(gf, Wf_a, bf_a.reshape(1, 1), 2000)
    eef = eef.reshape(E_F2F)
    numfp = _sc_scatter(wuef, fdst, zeros, N_FG, 64,
                        lambda wid, i: wid * 512 + i * 64,
                        lambda wid: jnp.where(wid < NW - 1, 8, 2))
    sfp = _sc_sdenom(eef, fdst, N_FG, 64,
                     lambda wid, i: wid * 512 + i * 64,
                     lambda wid: jnp.where(wid < NW - 1, 8, 2))
    conv_uff = _fg_upd(numfp, sfp, uff, Wf_n, bf_n.reshape(1, H))
    return (uaf, ubf, conv_uff)


# speedup vs baseline: 6.1660x; 1.0768x over previous
import functools

import jax
import jax.numpy as jnp
from jax import lax
from jax.experimental import pallas as pl
from jax.experimental.pallas import tpu as pltpu
from jax.experimental.pallas import tpu_sc as plsc

N_ATOM = 10000
E_A2A = 320000
N_FG = 2000
E_A2F = 10000
E_F2F = 16000
D = 128
DE = 16
H = 128
UFF = H + D

NC = 2
NS = 16
NW = NC * NS


def _mesh():
    return plsc.VectorSubcoreMesh(core_axis_name="c", subcore_axis_name="s")


def _zero_acc(zeros_hbm, acc_sh, s, n_rows):
    nblk = n_rows // 200
    full, extra = nblk // NS, nblk % NS
    nit = jnp.where(s < extra, full + 1, full)

    def zb(k, _):
        pltpu.sync_copy(zeros_hbm, acc_sh.at[pl.ds((k * NS + s) * 200, 200)])
        return 0

    lax.fori_loop(0, nit, zb, 0)


def _proj_node_body(af_ref, ws_ref, wd_ref, be_ref, ps_ref, pd_ref):
    a = af_ref[...]
    ps_ref[...] = jnp.dot(a, ws_ref[...], preferred_element_type=jnp.float32)
    pd_ref[...] = (jnp.dot(a, wd_ref[...], preferred_element_type=jnp.float32)
                   + be_ref[...])


def _proj_node(af, ws, wd, be2d, n_rows, blk):
    grid = (n_rows // blk,)
    return pl.pallas_call(
        _proj_node_body,
        grid=grid,
        in_specs=[
            pl.BlockSpec((blk, af.shape[1]), lambda i: (i, 0)),
            pl.BlockSpec(ws.shape, lambda i: (0, 0)),
            pl.BlockSpec(wd.shape, lambda i: (0, 0)),
            pl.BlockSpec(be2d.shape, lambda i: (0, 0)),
        ],
        out_specs=[
            pl.BlockSpec((blk, H), lambda i: (i, 0)),
            pl.BlockSpec((blk, H), lambda i: (i, 0)),
        ],
        out_shape=[
            jax.ShapeDtypeStruct((n_rows, H), jnp.float32),
            jax.ShapeDtypeStruct((n_rows, H), jnp.float32),
        ],
    )(af, ws, wd, be2d)


def _sc_gather_add_body(epw, sub, nsub, ps_hbm, pd_hbm, src_hbm, dst_hbm,
                        g_hbm, si_v, di_v, gs_v, gd_v, sem1, sem2):
    c = lax.axis_index("c")
    s = lax.axis_index("s")
    wid = c * NS + s
    ch = sub * nsub
    nit = epw // ch

    def body(i, _):
        base = wid * epw + i * ch
        pltpu.sync_copy(src_hbm.at[pl.ds(base, ch)], si_v)
        pltpu.sync_copy(dst_hbm.at[pl.ds(base, ch)], di_v)
        for j in range(nsub):
            sl = pl.ds(j * sub, sub)
            cp1 = pltpu.async_copy(ps_hbm.at[si_v.at[sl]], gs_v.at[sl], sem1)
            cp2 = pltpu.async_copy(pd_hbm.at[di_v.at[sl]], gd_v.at[sl], sem2)
            cp1.wait()
            cp2.wait()

        def add_row(r, _):
            for cg in range(H // 16):
                sl = pl.ds(cg * 16, 16)
                gs_v[r, sl] = gs_v[r, sl] + gd_v[r, sl]
            return 0

        lax.fori_loop(0, ch, add_row, 0)
        pltpu.sync_copy(gs_v, g_hbm.at[pl.ds(base, ch)])
        return 0

    lax.fori_loop(0, nit, body, 0)


def _sc_gather_add(ps, pd, src, dst, n_edges, sub, nsub):
    epw = n_edges // NW
    ch = sub * nsub
    body = functools.partial(_sc_gather_add_body, epw, sub, nsub)
    k = pl.kernel(
        body,
        mesh=_mesh(),
        out_type=jax.ShapeDtypeStruct((n_edges, H), jnp.float32),
        scratch_types=[
            pltpu.VMEM((ch,), jnp.int32),
            pltpu.VMEM((ch,), jnp.int32),
            pltpu.VMEM((ch, H), jnp.float32),
            pltpu.VMEM((ch, H), jnp.float32),
            pltpu.SemaphoreType.DMA,
            pltpu.SemaphoreType.DMA,
        ],
    )
    return k(ps, pd, src, dst)


def _edge_act_body(g_ref, bfe_ref, we_ref, wa_ref, ba_ref, ue_ref, ee_ref,
                   wue_ref):
    pre = g_ref[...] + jnp.dot(bfe_ref[...], we_ref[...],
                               preferred_element_type=jnp.float32)
    ue = jnp.maximum(pre, 0.0)
    ue_ref[...] = ue
    logit = jnp.dot(ue, wa_ref[...], preferred_element_type=jnp.float32)
    ee = jnp.exp(logit + ba_ref[...])
    ee_ref[...] = ee
    wue_ref[...] = ue * ee


def _edge_act(g, bf, we_e, wa, ba2d, blk):
    n = g.shape[0]
    grid = (n // blk,)
    return pl.pallas_call(
        _edge_act_body,
        grid=grid,
        in_specs=[
            pl.BlockSpec((blk, H), lambda i: (i, 0)),
            pl.BlockSpec((blk, DE), lambda i: (i, 0)),
            pl.BlockSpec(we_e.shape, lambda i: (0, 0)),
            pl.BlockSpec(wa.shape, lambda i: (0, 0)),
            pl.BlockSpec(ba2d.shape, lambda i: (0, 0)),
        ],
        out_specs=[
            pl.BlockSpec((blk, H), lambda i: (i, 0)),
            pl.BlockSpec((blk, 1), lambda i: (i, 0)),
            pl.BlockSpec((blk, H), lambda i: (i, 0)),
        ],
        out_shape=[
            jax.ShapeDtypeStruct((n, H), jnp.float32),
            jax.ShapeDtypeStruct((n, 1), jnp.float32),
            jax.ShapeDtypeStruct((n, H), jnp.float32),
        ],
    )(g, bf, we_e, wa, ba2d)


def _sc_scatter_body(n_nodes, ch, base_of, nit_of, wue_hbm, dst_hbm,
                     z_hbm, nump_hbm, w_v, di_v, acc_sh, sem):
    c = lax.axis_index("c")
    s = lax.axis_index("s")
    wid = c * NS + s
    _zero_acc(z_hbm, acc_sh, s, n_nodes)
    plsc.subcore_barrier()

    def body(i, _):
        base = base_of(wid, i)
        pltpu.sync_copy(wue_hbm.at[pl.ds(base, ch)], w_v)
        pltpu.sync_copy(dst_hbm.at[pl.ds(base, ch)], di_v)
        pltpu.sync_copy(w_v, acc_sh.at[di_v], add=True)
        return 0

    lax.fori_loop(0, nit_of(wid), body, 0)
    plsc.subcore_barrier()

    @pl.when(s == 0)
    def _():
        pltpu.sync_copy(acc_sh, nump_hbm.at[c])


def _sc_scatter(wue, dst, zeros, n_nodes, ch, base_of, nit_of):
    body = functools.partial(_sc_scatter_body, n_nodes, ch, base_of, nit_of)
    k = pl.kernel(
        body,
        mesh=_mesh(),
        out_type=jax.ShapeDtypeStruct((NC, n_nodes, H), jnp.float32),
        scratch_types=[
            pltpu.VMEM((ch, H), jnp.float32),
            pltpu.VMEM((ch,), jnp.int32),
            pltpu.VMEM_SHARED((n_nodes, H), jnp.float32),
            pltpu.SemaphoreType.DMA,
        ],
    )
    return k(wue, dst, zeros)


def _sc_scatter_a2a_body(wue_hbm, dst_hbm, z_hbm, nump_hbm,
                         w_v, di_v, w16_v, di16_v, acc_sh, sem1, sem2):
    c = lax.axis_index("c")
    s = lax.axis_index("s")
    wid = c * NS + s
    epw = E_A2A // NW
    _zero_acc(z_hbm, acc_sh, s, N_ATOM)
    plsc.subcore_barrier()

    def body(i, _):
        base = wid * epw + i * 128
        cp1 = pltpu.async_copy(wue_hbm.at[pl.ds(base, 128)], w_v, sem1)
        cp2 = pltpu.async_copy(dst_hbm.at[pl.ds(base, 128)], di_v, sem2)
        cp1.wait()
        cp2.wait()
        pltpu.sync_copy(w_v, acc_sh.at[di_v], add=True)
        return 0

    lax.fori_loop(0, epw // 128, body, 0)
    tbase = wid * epw + (epw // 128) * 128
    pltpu.sync_copy(wue_hbm.at[pl.ds(tbase, 16)], w16_v)
    pltpu.sync_copy(dst_hbm.at[pl.ds(tbase, 16)], di16_v)
    pltpu.sync_copy(w16_v, acc_sh.at[di16_v], add=True)
    plsc.subcore_barrier()

    @pl.when(s == 0)
    def _():
        pltpu.sync_copy(acc_sh, nump_hbm.at[c])


def _sc_scatter_a2a(wue, dst, zeros):
    k = pl.kernel(
        _sc_scatter_a2a_body,
        mesh=_mesh(),
        out_type=jax.ShapeDtypeStruct((NC, N_ATOM, H), jnp.float32),
        scratch_types=[
            pltpu.VMEM((128, H), jnp.float32),
            pltpu.VMEM((128,), jnp.int32),
            pltpu.VMEM((16, H), jnp.float32),
            pltpu.VMEM((16,), jnp.int32),
            pltpu.VMEM_SHARED((N_ATOM, H), jnp.float32),
            pltpu.SemaphoreType.DMA,
            pltpu.SemaphoreType.DMA,
        ],
    )
    return k(wue, dst, zeros)


def _sc_sdenom_body(n_nodes, ch, base_of, nit_of, ee_hbm, dst_hbm, sp_hbm,
                    ee_v, di_v, sacc_v):
    c = lax.axis_index("c")
    s = lax.axis_index("s")
    wid = c * NS + s

    def zs(i, _):
        sacc_v[pl.ds(i * 16, 16)] = jnp.zeros((16,), jnp.float32)
        return 0

    lax.fori_loop(0, n_nodes // 16, zs, 0)

    def body(i, _):
        base = base_of(wid, i)
        pltpu.sync_copy(ee_hbm.at[pl.ds(base, ch)], ee_v)
        pltpu.sync_copy(dst_hbm.at[pl.ds(base, ch)], di_v)
        for g in range(ch // 16):
            sl = pl.ds(g * 16, 16)
            plsc.addupdate_scatter(sacc_v, [di_v[sl]], ee_v[sl])
        return 0

    lax.fori_loop(0, nit_of(wid), body, 0)
    pltpu.sync_copy(sacc_v, sp_hbm.at[wid])


def _sc_sdenom(ee, dst, n_nodes, ch, base_of, nit_of):
    body = functools.partial(_sc_sdenom_body, n_nodes, ch, base_of, nit_of)
    k = pl.kernel(
        body,
        mesh=_mesh(),
        out_type=jax.ShapeDtypeStruct((NW, n_nodes), jnp.float32),
        scratch_types=[
            pltpu.VMEM((ch,), jnp.float32),
            pltpu.VMEM((ch,), jnp.int32),
            pltpu.VMEM((n_nodes,), jnp.float32),
        ],
        compiler_params=pltpu.CompilerParams(needs_layout_passes=False),
    )
    return k(ee, dst)


def _node_upd_body(np_ref, sp_ref, x_ref, wx_ref, wa_ref, b_ref, out_ref):
    num = np_ref[0] + np_ref[1]
    s = jnp.sum(sp_ref[...], axis=0)
    agg = num / (s[:, None] + 1e-9)
    out_ref[...] = jnp.maximum(
        jnp.dot(x_ref[...], wx_ref[...], preferred_element_type=jnp.float32)
        + jnp.dot(agg, wa_ref[...], preferred_element_type=jnp.float32)
        + b_ref[...], 0.0)


def _node_upd(nump, sp, x, wx, wa, b2d):
    n = x.shape[0]
    return pl.pallas_call(
        _node_upd_body,
        out_shape=jax.ShapeDtypeStruct((n, H), jnp.float32),
    )(nump, sp, x, wx, wa, b2d)


def _sc_a2f_body(uaf_hbm, src_hbm, dst_hbm, z_hbm, aggp_hbm,
                 si_v, di_v, rows_v, acc_sh, sem):
    c = lax.axis_index("c")
    s = lax.axis_index("s")
    wid = c * NS + s
    _zero_acc(z_hbm, acc_sh, s, N_FG)
    plsc.subcore_barrier()

    nit = jnp.where(wid < NW - 1, 4, 1)

    def body(i, _):
        base = wid * 320 + i * 80
        pltpu.sync_copy(src_hbm.at[pl.ds(base, 80)], si_v)
        pltpu.sync_copy(dst_hbm.at[pl.ds(base, 80)], di_v)
        pltpu.async_copy(uaf_hbm.at[si_v], rows_v, sem).wait()
        pltpu.sync_copy(rows_v, acc_sh.at[di_v], add=True)
        return 0

    lax.fori_loop(0, nit, body, 0)
    plsc.subcore_barrier()

    @pl.when(s == 0)
    def _():
        pltpu.sync_copy(acc_sh, aggp_hbm.at[c])


def _sc_a2f(uaf, src, dst, zeros):
    k = pl.kernel(
        _sc_a2f_body,
        mesh=_mesh(),
        out_type=jax.ShapeDtypeStruct((NC, N_FG, H), jnp.float32),
        scratch_types=[
            pltpu.VMEM((80,), jnp.int32),
            pltpu.VMEM((80,), jnp.int32),
            pltpu.VMEM((80, H), jnp.float32),
            pltpu.VMEM_SHARED((N_FG, H), jnp.float32),
            pltpu.SemaphoreType.DMA,
        ],
    )
    return k(uaf, src, dst, zeros)


def _fg_proj_body(aggp_ref, ff_ref, wf_ref, bfe_ref, uff_ref, pfs_ref, pfd_ref):
    aggm = aggp_ref[0] + aggp_ref[1]
    uff = jnp.concatenate([aggm, ff_ref[...]], axis=-1)
    uff_ref[...] = uff
    pfs_ref[...] = jnp.dot(uff, wf_ref[0:UFF, :],
                           preferred_element_type=jnp.float32)
    pfd_ref[...] = (jnp.dot(uff, wf_ref[UFF:2 * UFF, :],
                            preferred_element_type=jnp.float32)
                    + bfe_ref[...])


def _fg_proj(aggp, ff, wf_e, bfe2d):
    return pl.pallas_call(
        _fg_proj_body,
        out_shape=[
            jax.ShapeDtypeStruct((N_FG, UFF), jnp.float32),
            jax.ShapeDtypeStruct((N_FG, H), jnp.float32),
            jax.ShapeDtypeStruct((N_FG, H), jnp.float32),
        ],
    )(aggp, ff, wf_e, bfe2d)


def _sc_f2f_gather_body(ps_hbm, pd_hbm, src_hbm, dst_hbm, g_hbm,
                        si_v, di_v, gs_v, gd_v, sem1, sem2):
    c = lax.axis_index("c")
    s = lax.axis_index("s")
    wid = c * NS + s
    nit = jnp.where(wid < NW - 1, 8, 2)

    def body(i, _):
        base = wid * 512 + i * 64
        pltpu.sync_copy(src_hbm.at[pl.ds(base, 64)], si_v)
        pltpu.sync_copy(dst_hbm.at[pl.ds(base, 64)], di_v)
        cp1 = pltpu.async_copy(ps_hbm.at[si_v], gs_v, sem1)
        cp2 = pltpu.async_copy(pd_hbm.at[di_v], gd_v, sem2)
        cp1.wait()
        cp2.wait()

        def add_row(r, _):
            for cg in range(H // 16):
                sl = pl.ds(cg * 16, 16)
                gs_v[r, sl] = gs_v[r, sl] + gd_v[r, sl]
            return 0

        lax.fori_loop(0, 64, add_row, 0)
        pltpu.sync_copy(gs_v, g_hbm.at[pl.ds(base, 64)])
        return 0

    lax.fori_loop(0, nit, body, 0)


def _sc_f2f_gather(pfs, pfd, src, dst):
    k = pl.kernel(
        _sc_f2f_gather_body,
        mesh=_mesh(),
        out_type=jax.ShapeDtypeStruct((E_F2F, H), jnp.float32),
        scratch_types=[
            pltpu.VMEM((64,), jnp.int32),
            pltpu.VMEM((64,), jnp.int32),
            pltpu.VMEM((64, H), jnp.float32),
            pltpu.VMEM((64, H), jnp.float32),
            pltpu.SemaphoreType.DMA,
            pltpu.SemaphoreType.DMA,
        ],
    )
    return k(pfs, pfd, src, dst)


def _f2f_act_body(g_ref, wa_ref, ba_ref, ue_ref, ee_ref, wue_ref):
    ue = jnp.maximum(g_ref[...], 0.0)
    ue_ref[...] = ue
    logit = jnp.dot(ue, wa_ref[...], preferred_element_type=jnp.float32)
    ee = jnp.exp(logit + ba_ref[...])
    ee_ref[...] = ee
    wue_ref[...] = ue * ee


def _f2f_act(g, wa, ba2d, blk):
    n = g.shape[0]
    return pl.pallas_call(
        _f2f_act_body,
        grid=(n // blk,),
        in_specs=[
            pl.BlockSpec((blk, H), lambda i: (i, 0)),
            pl.BlockSpec(wa.shape, lambda i: (0, 0)),
            pl.BlockSpec(ba2d.shape, lambda i: (0, 0)),
        ],
        out_specs=[
            pl.BlockSpec((blk, H), lambda i: (i, 0)),
            pl.BlockSpec((blk, 1), lambda i: (i, 0)),
            pl.BlockSpec((blk, H), lambda i: (i, 0)),
        ],
        out_shape=[
            jax.ShapeDtypeStruct((n, H), jnp.float32),
            jax.ShapeDtypeStruct((n, 1), jnp.float32),
            jax.ShapeDtypeStruct((n, H), jnp.float32),
        ],
    )(g, wa, ba2d)


def _fg_upd_body(np_ref, sp_ref, uff_ref, wf_ref, b_ref, out_ref):
    num = np_ref[0] + np_ref[1]
    s = jnp.sum(sp_ref[...], axis=0)
    agg = num / (s[:, None] + 1e-9)
    uff = uff_ref[...]
    out_ref[...] = jnp.maximum(
        jnp.dot(uff, wf_ref[0:UFF, :], preferred_element_type=jnp.float32)
        + jnp.dot(agg, wf_ref[UFF:UFF + H, :],
                  preferred_element_type=jnp.float32)
        + b_ref[...], 0.0)


def _fg_upd(nump, sp, uff, wf_n, b2d):
    return pl.pallas_call(
        _fg_upd_body,
        out_shape=jax.ShapeDtypeStruct((N_FG, H), jnp.float32),
    )(nump, sp, uff, wf_n, b2d)


def kernel(af, bf, ff, W_e, b_e, W_a, b_a, W_n, b_n, Wf_e, bf_e, Wf_a, bf_a,
           Wf_n, bf_n, a2a_edge_index, a2f_src, a2f_dst, f2f_edge_index):
    src = a2a_edge_index[0].astype(jnp.int32)
    dst = a2a_edge_index[1].astype(jnp.int32)
    fsrc = f2f_edge_index[0].astype(jnp.int32)
    fdst = f2f_edge_index[1].astype(jnp.int32)
    asrc = a2f_src.astype(jnp.int32)
    adst = a2f_dst.astype(jnp.int32)

    ps, pd = _proj_node(af, W_e[:D], W_e[D:2 * D], b_e.reshape(1, H),
                        N_ATOM, 1000)
    g = _sc_gather_add(ps, pd, src, dst, E_A2A, 80, 5)
    ubf, ee, wue = _edge_act(g, bf, W_e[2 * D:], W_a, b_a.reshape(1, 1), 2000)
    ee = ee.reshape(E_A2A)
    zeros = jnp.zeros((200, H), jnp.float32)
    epw = E_A2A // NW
    nump = _sc_scatter_a2a(wue, dst, zeros)
    sp = _sc_sdenom(ee, dst, N_ATOM, 80,
                    lambda wid, i: wid * epw + i * 80,
                    lambda wid: epw // 80)
    uaf = _node_upd(nump, sp, af, W_n[:D], W_n[D:], b_n.reshape(1, H))
    aggp = _sc_a2f(uaf, asrc, adst, zeros)
    uff, pfs, pfd = _fg_proj(aggp, ff, Wf_e, bf_e.reshape(1, H))
    gf = _sc_f2f_gather(pfs, pfd, fsrc, fdst)
    uef, eef, wuef = _f2f_act(gf, Wf_a, bf_a.reshape(1, 1), 2000)
    eef = eef.reshape(E_F2F)
    numfp = _sc_scatter(wuef, fdst, zeros, N_FG, 64,
                        lambda wid, i: wid * 512 + i * 64,
                        lambda wid: jnp.where(wid < NW - 1, 8, 2))
    sfp = _sc_sdenom(eef, fdst, N_FG, 64,
                     lambda wid, i: wid * 512 + i * 64,
                     lambda wid: jnp.where(wid < NW - 1, 8, 2))
    conv_uff = _fg_upd(numfp, sfp, uff, Wf_n, bf_n.reshape(1, H))
    return (uaf, ubf, conv_uff)


# speedup vs baseline: 6.7606x; 1.0964x over previous
import functools

import jax
import jax.numpy as jnp
from jax import lax
from jax.experimental import pallas as pl
from jax.experimental.pallas import tpu as pltpu
from jax.experimental.pallas import tpu_sc as plsc

N_ATOM = 10000
E_A2A = 320000
N_FG = 2000
E_A2F = 10000
E_F2F = 16000
D = 128
DE = 16
H = 128
UFF = H + D

NC = 2
NS = 16
NW = NC * NS


def _mesh():
    return plsc.VectorSubcoreMesh(core_axis_name="c", subcore_axis_name="s")


def _zero_acc(zeros_hbm, acc_sh, s, n_rows):
    nblk = n_rows // 200
    full, extra = nblk // NS, nblk % NS
    nit = jnp.where(s < extra, full + 1, full)

    def zb(k, _):
        pltpu.sync_copy(zeros_hbm, acc_sh.at[pl.ds((k * NS + s) * 200, 200)])
        return 0

    lax.fori_loop(0, nit, zb, 0)


def _proj_node_body(af_ref, ws_ref, wd_ref, be_ref, ps_ref, pd_ref):
    a = af_ref[...]
    ps_ref[...] = jnp.dot(a, ws_ref[...], preferred_element_type=jnp.float32)
    pd_ref[...] = (jnp.dot(a, wd_ref[...], preferred_element_type=jnp.float32)
                   + be_ref[...])


def _proj_node(af, ws, wd, be2d, n_rows, blk):
    grid = (n_rows // blk,)
    return pl.pallas_call(
        _proj_node_body,
        grid=grid,
        in_specs=[
            pl.BlockSpec((blk, af.shape[1]), lambda i: (i, 0)),
            pl.BlockSpec(ws.shape, lambda i: (0, 0)),
            pl.BlockSpec(wd.shape, lambda i: (0, 0)),
            pl.BlockSpec(be2d.shape, lambda i: (0, 0)),
        ],
        out_specs=[
            pl.BlockSpec((blk, H), lambda i: (i, 0)),
            pl.BlockSpec((blk, H), lambda i: (i, 0)),
        ],
        out_shape=[
            jax.ShapeDtypeStruct((n_rows, H), jnp.float32),
            jax.ShapeDtypeStruct((n_rows, H), jnp.float32),
        ],
    )(af, ws, wd, be2d)


def _sc_gather_add_body(epw, sub, nsub, ps_hbm, pd_hbm, src_hbm, dst_hbm,
                        g_hbm, si_v, di_v, gs_v, gd_v, sem1, sem2):
    c = lax.axis_index("c")
    s = lax.axis_index("s")
    wid = c * NS + s
    ch = sub * nsub
    nit = epw // ch

    def body(i, _):
        base = wid * epw + i * ch
        pltpu.sync_copy(src_hbm.at[pl.ds(base, ch)], si_v)
        pltpu.sync_copy(dst_hbm.at[pl.ds(base, ch)], di_v)
        for j in range(nsub):
            sl = pl.ds(j * sub, sub)
            cp1 = pltpu.async_copy(ps_hbm.at[si_v.at[sl]], gs_v.at[sl], sem1)
            cp2 = pltpu.async_copy(pd_hbm.at[di_v.at[sl]], gd_v.at[sl], sem2)
            cp1.wait()
            cp2.wait()

        def add_row(r, _):
            for cg in range(H // 16):
                sl = pl.ds(cg * 16, 16)
                gs_v[r, sl] = gs_v[r, sl] + gd_v[r, sl]
            return 0

        lax.fori_loop(0, ch, add_row, 0)
        pltpu.sync_copy(gs_v, g_hbm.at[pl.ds(base, ch)])
        return 0

    lax.fori_loop(0, nit, body, 0)


def _sc_gather_add(ps, pd, src, dst, n_edges, sub, nsub):
    epw = n_edges // NW
    ch = sub * nsub
    body = functools.partial(_sc_gather_add_body, epw, sub, nsub)
    k = pl.kernel(
        body,
        mesh=_mesh(),
        out_type=jax.ShapeDtypeStruct((n_edges, H), jnp.float32),
        scratch_types=[
            pltpu.VMEM((ch,), jnp.int32),
            pltpu.VMEM((ch,), jnp.int32),
            pltpu.VMEM((ch, H), jnp.float32),
            pltpu.VMEM((ch, H), jnp.float32),
            pltpu.SemaphoreType.DMA,
            pltpu.SemaphoreType.DMA,
        ],
    )
    return k(ps, pd, src, dst)


def _edge_act_body(g_ref, bfe_ref, we_ref, wa_ref, ba_ref, ue_ref, ee_ref,
                   wue_ref):
    pre = g_ref[...] + jnp.dot(bfe_ref[...], we_ref[...],
                               preferred_element_type=jnp.float32)
    ue = jnp.maximum(pre, 0.0)
    ue_ref[...] = ue
    logit = jnp.dot(ue, wa_ref[...], preferred_element_type=jnp.float32)
    ee = jnp.exp(logit + ba_ref[...])
    ee_ref[...] = ee
    wue_ref[...] = ue * ee


def _edge_act(g, bf, we_e, wa, ba2d, blk):
    n = g.shape[0]
    grid = (n // blk,)
    return pl.pallas_call(
        _edge_act_body,
        grid=grid,
        in_specs=[
            pl.BlockSpec((blk, H), lambda i: (i, 0)),
            pl.BlockSpec((blk, DE), lambda i: (i, 0)),
            pl.BlockSpec(we_e.shape, lambda i: (0, 0)),
            pl.BlockSpec(wa.shape, lambda i: (0, 0)),
            pl.BlockSpec(ba2d.shape, lambda i: (0, 0)),
        ],
        out_specs=[
            pl.BlockSpec((blk, H), lambda i: (i, 0)),
            pl.BlockSpec((blk, 1), lambda i: (i, 0)),
            pl.BlockSpec((blk, H), lambda i: (i, 0)),
        ],
        out_shape=[
            jax.ShapeDtypeStruct((n, H), jnp.float32),
            jax.ShapeDtypeStruct((n, 1), jnp.float32),
            jax.ShapeDtypeStruct((n, H), jnp.float32),
        ],
    )(g, bf, we_e, wa, ba2d)


def _sc_scatter_body(n_nodes, ch, base_of, nit_of, wue_hbm, dst_hbm,
                     z_hbm, nump_hbm, w_v, di_v, acc_sh, sem):
    c = lax.axis_index("c")
    s = lax.axis_index("s")
    wid = c * NS + s
    _zero_acc(z_hbm, acc_sh, s, n_nodes)
    plsc.subcore_barrier()

    def body(i, _):
        base = base_of(wid, i)
        pltpu.sync_copy(wue_hbm.at[pl.ds(base, ch)], w_v)
        pltpu.sync_copy(dst_hbm.at[pl.ds(base, ch)], di_v)
        pltpu.sync_copy(w_v, acc_sh.at[di_v], add=True)
        return 0

    lax.fori_loop(0, nit_of(wid), body, 0)
    plsc.subcore_barrier()

    @pl.when(s == 0)
    def _():
        pltpu.sync_copy(acc_sh, nump_hbm.at[c])


def _sc_scatter(wue, dst, zeros, n_nodes, ch, base_of, nit_of):
    body = functools.partial(_sc_scatter_body, n_nodes, ch, base_of, nit_of)
    k = pl.kernel(
        body,
        mesh=_mesh(),
        out_type=jax.ShapeDtypeStruct((NC, n_nodes, H), jnp.float32),
        scratch_types=[
            pltpu.VMEM((ch, H), jnp.float32),
            pltpu.VMEM((ch,), jnp.int32),
            pltpu.VMEM_SHARED((n_nodes, H), jnp.float32),
            pltpu.SemaphoreType.DMA,
        ],
    )
    return k(wue, dst, zeros)


def _sc_scatter_a2a_body(wue_hbm, dst_hbm, z_hbm, nump_hbm,
                         w_v, di_v, w16_v, di16_v, acc_sh, sem1, sem2):
    c = lax.axis_index("c")
    s = lax.axis_index("s")
    wid = c * NS + s
    epw = E_A2A // NW
    _zero_acc(z_hbm, acc_sh, s, N_ATOM)
    plsc.subcore_barrier()

    def body(i, _):
        base = wid * epw + i * 128
        cp1 = pltpu.async_copy(wue_hbm.at[pl.ds(base, 128)], w_v, sem1)
        cp2 = pltpu.async_copy(dst_hbm.at[pl.ds(base, 128)], di_v, sem2)
        cp1.wait()
        cp2.wait()
        pltpu.sync_copy(w_v, acc_sh.at[di_v], add=True)
        return 0

    lax.fori_loop(0, epw // 128, body, 0)
    tbase = wid * epw + (epw // 128) * 128
    pltpu.sync_copy(wue_hbm.at[pl.ds(tbase, 16)], w16_v)
    pltpu.sync_copy(dst_hbm.at[pl.ds(tbase, 16)], di16_v)
    pltpu.sync_copy(w16_v, acc_sh.at[di16_v], add=True)
    plsc.subcore_barrier()

    @pl.when(s == 0)
    def _():
        pltpu.sync_copy(acc_sh, nump_hbm.at[c])


def _sc_scatter_a2a(wue, dst, zeros):
    k = pl.kernel(
        _sc_scatter_a2a_body,
        mesh=_mesh(),
        out_type=jax.ShapeDtypeStruct((NC, N_ATOM, H), jnp.float32),
        scratch_types=[
            pltpu.VMEM((128, H), jnp.float32),
            pltpu.VMEM((128,), jnp.int32),
            pltpu.VMEM((16, H), jnp.float32),
            pltpu.VMEM((16,), jnp.int32),
            pltpu.VMEM_SHARED((N_ATOM, H), jnp.float32),
            pltpu.SemaphoreType.DMA,
            pltpu.SemaphoreType.DMA,
        ],
    )
    return k(wue, dst, zeros)


def _sc_sdenom_body(n_nodes, ch, base_of, nit_of, ee_hbm, dst_hbm, sp_hbm,
                    ee_v, di_v, sacc_v):
    c = lax.axis_index("c")
    s = lax.axis_index("s")
    wid = c * NS + s

    def zs(i, _):
        sacc_v[pl.ds(i * 16, 16)] = jnp.zeros((16,), jnp.float32)
        return 0

    lax.fori_loop(0, n_nodes // 16, zs, 0)

    def body(i, _):
        base = base_of(wid, i)
        pltpu.sync_copy(ee_hbm.at[pl.ds(base, ch)], ee_v)
        pltpu.sync_copy(dst_hbm.at[pl.ds(base, ch)], di_v)
        for g in range(ch // 16):
            sl = pl.ds(g * 16, 16)
            plsc.addupdate_scatter(sacc_v, [di_v[sl]], ee_v[sl])
        return 0

    lax.fori_loop(0, nit_of(wid), body, 0)
    pltpu.sync_copy(sacc_v, sp_hbm.at[wid])


def _sc_sdenom(ee, dst, n_nodes, ch, base_of, nit_of):
    body = functools.partial(_sc_sdenom_body, n_nodes, ch, base_of, nit_of)
    k = pl.kernel(
        body,
        mesh=_mesh(),
        out_type=jax.ShapeDtypeStruct((NW, n_nodes), jnp.float32),
        scratch_types=[
            pltpu.VMEM((ch,), jnp.float32),
            pltpu.VMEM((ch,), jnp.int32),
            pltpu.VMEM((n_nodes,), jnp.float32),
        ],
        compiler_params=pltpu.CompilerParams(needs_layout_passes=False),
    )
    return k(ee, dst)


def _node_upd_body(np_ref, sp_ref, x_ref, wx_ref, wa_ref, b_ref, out_ref):
    num = np_ref[0] + np_ref[1]
    s = jnp.sum(sp_ref[...], axis=0)
    agg = num / (s[:, None] + 1e-9)
    out_ref[...] = jnp.maximum(
        jnp.dot(x_ref[...], wx_ref[...], preferred_element_type=jnp.float32)
        + jnp.dot(agg, wa_ref[...], preferred_element_type=jnp.float32)
        + b_ref[...], 0.0)


def _node_upd(nump, sp, x, wx, wa, b2d):
    n = x.shape[0]
    return pl.pallas_call(
        _node_upd_body,
        out_shape=jax.ShapeDtypeStruct((n, H), jnp.float32),
    )(nump, sp, x, wx, wa, b2d)


def _sc_a2f_body(uaf_hbm, src_hbm, dst_hbm, z_hbm, aggp_hbm,
                 si_v, di_v, rows_v, acc_sh, sem):
    c = lax.axis_index("c")
    s = lax.axis_index("s")
    wid = c * NS + s
    _zero_acc(z_hbm, acc_sh, s, N_FG)
    plsc.subcore_barrier()

    nit = jnp.where(wid < NW - 1, 4, 1)

    def body(i, _):
        base = wid * 320 + i * 80
        pltpu.sync_copy(src_hbm.at[pl.ds(base, 80)], si_v)
        pltpu.sync_copy(dst_hbm.at[pl.ds(base, 80)], di_v)
        pltpu.async_copy(uaf_hbm.at[si_v], rows_v, sem).wait()
        pltpu.sync_copy(rows_v, acc_sh.at[di_v], add=True)
        return 0

    lax.fori_loop(0, nit, body, 0)
    plsc.subcore_barrier()

    @pl.when(s == 0)
    def _():
        pltpu.sync_copy(acc_sh, aggp_hbm.at[c])


def _sc_a2f(uaf, src, dst, zeros):
    k = pl.kernel(
        _sc_a2f_body,
        mesh=_mesh(),
        out_type=jax.ShapeDtypeStruct((NC, N_FG, H), jnp.float32),
        scratch_types=[
            pltpu.VMEM((80,), jnp.int32),
            pltpu.VMEM((80,), jnp.int32),
            pltpu.VMEM((80, H), jnp.float32),
            pltpu.VMEM_SHARED((N_FG, H), jnp.float32),
            pltpu.SemaphoreType.DMA,
        ],
    )
    return k(uaf, src, dst, zeros)


def _fg_proj_body(aggp_ref, ff_ref, wf_ref, bfe_ref, uff_ref, pfs_ref, pfd_ref):
    aggm = aggp_ref[0] + aggp_ref[1]
    uff = jnp.concatenate([aggm, ff_ref[...]], axis=-1)
    uff_ref[...] = uff
    pfs_ref[...] = jnp.dot(uff, wf_ref[0:UFF, :],
                           preferred_element_type=jnp.float32)
    pfd_ref[...] = (jnp.dot(uff, wf_ref[UFF:2 * UFF, :],
                            preferred_element_type=jnp.float32)
                    + bfe_ref[...])


def _fg_proj(aggp, ff, wf_e, bfe2d):
    return pl.pallas_call(
        _fg_proj_body,
        out_shape=[
            jax.ShapeDtypeStruct((N_FG, UFF), jnp.float32),
            jax.ShapeDtypeStruct((N_FG, H), jnp.float32),
            jax.ShapeDtypeStruct((N_FG, H), jnp.float32),
        ],
    )(aggp, ff, wf_e, bfe2d)


def _sc_f2f_gather_body(ps_hbm, pd_hbm, src_hbm, dst_hbm, g_hbm,
                        si_v, di_v, gs_v, gd_v, sem1, sem2):
    c = lax.axis_index("c")
    s = lax.axis_index("s")
    wid = c * NS + s
    nit = jnp.where(wid < NW - 1, 8, 2)

    def body(i, _):
        base = wid * 512 + i * 64
        pltpu.sync_copy(src_hbm.at[pl.ds(base, 64)], si_v)
        pltpu.sync_copy(dst_hbm.at[pl.ds(base, 64)], di_v)
        cp1 = pltpu.async_copy(ps_hbm.at[si_v], gs_v, sem1)
        cp2 = pltpu.async_copy(pd_hbm.at[di_v], gd_v, sem2)
        cp1.wait()
        cp2.wait()

        def add_row(r, _):
            for cg in range(H // 16):
                sl = pl.ds(cg * 16, 16)
                gs_v[r, sl] = gs_v[r, sl] + gd_v[r, sl]
            return 0

        lax.fori_loop(0, 64, add_row, 0)
        pltpu.sync_copy(gs_v, g_hbm.at[pl.ds(base, 64)])
        return 0

    lax.fori_loop(0, nit, body, 0)


def _sc_f2f_gather(pfs, pfd, src, dst):
    k = pl.kernel(
        _sc_f2f_gather_body,
        mesh=_mesh(),
        out_type=jax.ShapeDtypeStruct((E_F2F, H), jnp.float32),
        scratch_types=[
            pltpu.VMEM((64,), jnp.int32),
            pltpu.VMEM((64,), jnp.int32),
            pltpu.VMEM((64, H), jnp.float32),
            pltpu.VMEM((64, H), jnp.float32),
            pltpu.SemaphoreType.DMA,
            pltpu.SemaphoreType.DMA,
        ],
    )
    return k(pfs, pfd, src, dst)


def _f2f_act_body(g_ref, wa_ref, ba_ref, ue_ref, ee_ref, wue_ref):
    ue = jnp.maximum(g_ref[...], 0.0)
    ue_ref[...] = ue
    logit = jnp.dot(ue, wa_ref[...], preferred_element_type=jnp.float32)
    ee = jnp.exp(logit + ba_ref[...])
    ee_ref[...] = ee
    wue_ref[...] = ue * ee


def _f2f_act(g, wa, ba2d, blk):
    n = g.shape[0]
    return pl.pallas_call(
        _f2f_act_body,
        grid=(n // blk,),
        in_specs=[
            pl.BlockSpec((blk, H), lambda i: (i, 0)),
            pl.BlockSpec(wa.shape, lambda i: (0, 0)),
            pl.BlockSpec(ba2d.shape, lambda i: (0, 0)),
        ],
        out_specs=[
            pl.BlockSpec((blk, H), lambda i: (i, 0)),
            pl.BlockSpec((blk, 1), lambda i: (i, 0)),
            pl.BlockSpec((blk, H), lambda i: (i, 0)),
        ],
        out_shape=[
            jax.ShapeDtypeStruct((n, H), jnp.float32),
            jax.ShapeDtypeStruct((n, 1), jnp.float32),
            jax.ShapeDtypeStruct((n, H), jnp.float32),
        ],
    )(g, wa, ba2d)


def _fg_upd_body(np_ref, sp_ref, uff_ref, wf_ref, b_ref, out_ref):
    num = np_ref[0] + np_ref[1]
    s = jnp.sum(sp_ref[...], axis=0)
    agg = num / (s[:, None] + 1e-9)
    uff = uff_ref[...]
    out_ref[...] = jnp.maximum(
        jnp.dot(uff, wf_ref[0:UFF, :], preferred_element_type=jnp.float32)
        + jnp.dot(agg, wf_ref[UFF:UFF + H, :],
                  preferred_element_type=jnp.float32)
        + b_ref[...], 0.0)


def _fg_upd(nump, sp, uff, wf_n, b2d):
    return pl.pallas_call(
        _fg_upd_body,
        out_shape=jax.ShapeDtypeStruct((N_FG, H), jnp.float32),
    )(nump, sp, uff, wf_n, b2d)


def kernel(af, bf, ff, W_e, b_e, W_a, b_a, W_n, b_n, Wf_e, bf_e, Wf_a, bf_a,
           Wf_n, bf_n, a2a_edge_index, a2f_src, a2f_dst, f2f_edge_index):
    src = a2a_edge_index[0].astype(jnp.int32)
    dst = a2a_edge_index[1].astype(jnp.int32)
    fsrc = f2f_edge_index[0].astype(jnp.int32)
    fdst = f2f_edge_index[1].astype(jnp.int32)
    asrc = a2f_src.astype(jnp.int32)
    adst = a2f_dst.astype(jnp.int32)

    ps, pd = _proj_node(af, W_e[:D], W_e[D:2 * D], b_e.reshape(1, H),
                        N_ATOM, 1000)
    g = _sc_gather_add(ps, pd, src, dst, E_A2A, 80, 5)
    ubf, ee, wue = _edge_act(g, bf, W_e[2 * D:], W_a, b_a.reshape(1, 1), 2000)
    ee = ee.reshape(E_A2A)
    zeros = jnp.zeros((200, H), jnp.float32)
    epw = E_A2A // NW
    nump = _sc_scatter_a2a(wue, dst, zeros)
    sp = _sc_sdenom(ee, dst, N_ATOM, 2000,
                    lambda wid, i: wid * epw + i * 2000,
                    lambda wid: epw // 2000)
    uaf = _node_upd(nump, sp, af, W_n[:D], W_n[D:], b_n.reshape(1, H))
    aggp = _sc_a2f(uaf, asrc, adst, zeros)
    uff, pfs, pfd = _fg_proj(aggp, ff, Wf_e, bf_e.reshape(1, H))
    gf = _sc_f2f_gather(pfs, pfd, fsrc, fdst)
    uef, eef, wuef = _f2f_act(gf, Wf_a, bf_a.reshape(1, 1), 2000)
    eef = eef.reshape(E_F2F)
    numfp = _sc_scatter(wuef, fdst, zeros, N_FG, 64,
                        lambda wid, i: wid * 512 + i * 64,
                        lambda wid: jnp.where(wid < NW - 1, 8, 2))
    sfp = _sc_sdenom(eef, fdst, N_FG, 64,
                     lambda wid, i: wid * 512 + i * 64,
                     lambda wid: jnp.where(wid < NW - 1, 8, 2))
    conv_uff = _fg_upd(numfp, sfp, uff, Wf_n, bf_n.reshape(1, H))
    return (uaf, ubf, conv_uff)


# speedup vs baseline: 7.3141x; 1.0819x over previous
import functools

import jax
import jax.numpy as jnp
from jax import lax
from jax.experimental import pallas as pl
from jax.experimental.pallas import tpu as pltpu
from jax.experimental.pallas import tpu_sc as plsc

N_ATOM = 10000
E_A2A = 320000
N_FG = 2000
E_A2F = 10000
E_F2F = 16000
D = 128
DE = 16
H = 128
UFF = H + D

NC = 2
NS = 16
NW = NC * NS


def _mesh():
    return plsc.VectorSubcoreMesh(core_axis_name="c", subcore_axis_name="s")


def _zero_acc(zeros_hbm, acc_sh, s, n_rows):
    nblk = n_rows // 200
    full, extra = nblk // NS, nblk % NS
    nit = jnp.where(s < extra, full + 1, full)

    def zb(k, _):
        pltpu.sync_copy(zeros_hbm, acc_sh.at[pl.ds((k * NS + s) * 200, 200)])
        return 0

    lax.fori_loop(0, nit, zb, 0)


def _proj_node_body(af_ref, ws_ref, wd_ref, be_ref, ps_ref, pd_ref):
    a = af_ref[...]
    ps_ref[...] = jnp.dot(a, ws_ref[...], preferred_element_type=jnp.float32)
    pd_ref[...] = (jnp.dot(a, wd_ref[...], preferred_element_type=jnp.float32)
                   + be_ref[...])


def _proj_node(af, ws, wd, be2d, n_rows, blk):
    grid = (n_rows // blk,)
    return pl.pallas_call(
        _proj_node_body,
        grid=grid,
        in_specs=[
            pl.BlockSpec((blk, af.shape[1]), lambda i: (i, 0)),
            pl.BlockSpec(ws.shape, lambda i: (0, 0)),
            pl.BlockSpec(wd.shape, lambda i: (0, 0)),
            pl.BlockSpec(be2d.shape, lambda i: (0, 0)),
        ],
        out_specs=[
            pl.BlockSpec((blk, H), lambda i: (i, 0)),
            pl.BlockSpec((blk, H), lambda i: (i, 0)),
        ],
        out_shape=[
            jax.ShapeDtypeStruct((n_rows, H), jnp.float32),
            jax.ShapeDtypeStruct((n_rows, H), jnp.float32),
        ],
    )(af, ws, wd, be2d)


def _sc_gather_add_body(epw, sub, nsub, ps_hbm, pd_hbm, src_hbm, dst_hbm,
                        g_hbm, si_v, di_v, gs_v, gd_v, sem1, sem2):
    c = lax.axis_index("c")
    s = lax.axis_index("s")
    wid = c * NS + s
    ch = sub * nsub
    nit = epw // ch

    def body(i, _):
        base = wid * epw + i * ch
        pltpu.sync_copy(src_hbm.at[pl.ds(base, ch)], si_v)
        pltpu.sync_copy(dst_hbm.at[pl.ds(base, ch)], di_v)
        def add_rows(j):
            def add_row(r, _):
                for cg in range(H // 16):
                    sl = pl.ds(cg * 16, 16)
                    gs_v[r, sl] = gs_v[r, sl] + gd_v[r, sl]
                return 0

            lax.fori_loop(j * sub, (j + 1) * sub, add_row, 0)

        def gather(j):
            sl = pl.ds(j * sub, sub)
            cp1 = pltpu.async_copy(ps_hbm.at[si_v.at[sl]], gs_v.at[sl], sem1)
            cp2 = pltpu.async_copy(pd_hbm.at[di_v.at[sl]], gd_v.at[sl], sem2)
            return cp1, cp2

        prev = gather(0)
        for j in range(1, nsub):
            cur = gather(j)
            prev[0].wait()
            prev[1].wait()
            add_rows(j - 1)
            prev = cur
        prev[0].wait()
        prev[1].wait()
        add_rows(nsub - 1)
        pltpu.sync_copy(gs_v, g_hbm.at[pl.ds(base, ch)])
        return 0

    lax.fori_loop(0, nit, body, 0)


def _sc_gather_add(ps, pd, src, dst, n_edges, sub, nsub):
    epw = n_edges // NW
    ch = sub * nsub
    body = functools.partial(_sc_gather_add_body, epw, sub, nsub)
    k = pl.kernel(
        body,
        mesh=_mesh(),
        out_type=jax.ShapeDtypeStruct((n_edges, H), jnp.float32),
        scratch_types=[
            pltpu.VMEM((ch,), jnp.int32),
            pltpu.VMEM((ch,), jnp.int32),
            pltpu.VMEM((ch, H), jnp.float32),
            pltpu.VMEM((ch, H), jnp.float32),
            pltpu.SemaphoreType.DMA,
            pltpu.SemaphoreType.DMA,
        ],
    )
    return k(ps, pd, src, dst)


def _edge_act_body(g_ref, bfe_ref, we_ref, wa_ref, ba_ref, ue_ref, ee_ref,
                   wue_ref):
    pre = g_ref[...] + jnp.dot(bfe_ref[...], we_ref[...],
                               preferred_element_type=jnp.float32)
    ue = jnp.maximum(pre, 0.0)
    ue_ref[...] = ue
    logit = jnp.dot(ue, wa_ref[...], preferred_element_type=jnp.float32)
    ee = jnp.exp(logit + ba_ref[...])
    ee_ref[...] = ee
    wue_ref[...] = ue * ee


def _edge_act(g, bf, we_e, wa, ba2d, blk):
    n = g.shape[0]
    grid = (n // blk,)
    return pl.pallas_call(
        _edge_act_body,
        grid=grid,
        in_specs=[
            pl.BlockSpec((blk, H), lambda i: (i, 0)),
            pl.BlockSpec((blk, DE), lambda i: (i, 0)),
            pl.BlockSpec(we_e.shape, lambda i: (0, 0)),
            pl.BlockSpec(wa.shape, lambda i: (0, 0)),
            pl.BlockSpec(ba2d.shape, lambda i: (0, 0)),
        ],
        out_specs=[
            pl.BlockSpec((blk, H), lambda i: (i, 0)),
            pl.BlockSpec((blk, 1), lambda i: (i, 0)),
            pl.BlockSpec((blk, H), lambda i: (i, 0)),
        ],
        out_shape=[
            jax.ShapeDtypeStruct((n, H), jnp.float32),
            jax.ShapeDtypeStruct((n, 1), jnp.float32),
            jax.ShapeDtypeStruct((n, H), jnp.float32),
        ],
    )(g, bf, we_e, wa, ba2d)


def _sc_scatter_body(n_nodes, ch, base_of, nit_of, wue_hbm, dst_hbm,
                     z_hbm, nump_hbm, w_v, di_v, acc_sh, sem):
    c = lax.axis_index("c")
    s = lax.axis_index("s")
    wid = c * NS + s
    _zero_acc(z_hbm, acc_sh, s, n_nodes)
    plsc.subcore_barrier()

    def body(i, _):
        base = base_of(wid, i)
        pltpu.sync_copy(wue_hbm.at[pl.ds(base, ch)], w_v)
        pltpu.sync_copy(dst_hbm.at[pl.ds(base, ch)], di_v)
        pltpu.sync_copy(w_v, acc_sh.at[di_v], add=True)
        return 0

    lax.fori_loop(0, nit_of(wid), body, 0)
    plsc.subcore_barrier()

    @pl.when(s == 0)
    def _():
        pltpu.sync_copy(acc_sh, nump_hbm.at[c])


def _sc_scatter(wue, dst, zeros, n_nodes, ch, base_of, nit_of):
    body = functools.partial(_sc_scatter_body, n_nodes, ch, base_of, nit_of)
    k = pl.kernel(
        body,
        mesh=_mesh(),
        out_type=jax.ShapeDtypeStruct((NC, n_nodes, H), jnp.float32),
        scratch_types=[
            pltpu.VMEM((ch, H), jnp.float32),
            pltpu.VMEM((ch,), jnp.int32),
            pltpu.VMEM_SHARED((n_nodes, H), jnp.float32),
            pltpu.SemaphoreType.DMA,
        ],
    )
    return k(wue, dst, zeros)


def _sc_scatter_a2a_body(wue_hbm, dst_hbm, z_hbm, nump_hbm,
                         w_v, di_v, w16_v, di16_v, acc_sh, sem1, sem2):
    c = lax.axis_index("c")
    s = lax.axis_index("s")
    wid = c * NS + s
    epw = E_A2A // NW
    _zero_acc(z_hbm, acc_sh, s, N_ATOM)
    plsc.subcore_barrier()

    def body(i, _):
        base = wid * epw + i * 128
        cp1 = pltpu.async_copy(wue_hbm.at[pl.ds(base, 128)], w_v, sem1)
        cp2 = pltpu.async_copy(dst_hbm.at[pl.ds(base, 128)], di_v, sem2)
        cp1.wait()
        cp2.wait()
        pltpu.sync_copy(w_v, acc_sh.at[di_v], add=True)
        return 0

    lax.fori_loop(0, epw // 128, body, 0)
    tbase = wid * epw + (epw // 128) * 128
    pltpu.sync_copy(wue_hbm.at[pl.ds(tbase, 16)], w16_v)
    pltpu.sync_copy(dst_hbm.at[pl.ds(tbase, 16)], di16_v)
    pltpu.sync_copy(w16_v, acc_sh.at[di16_v], add=True)
    plsc.subcore_barrier()

    @pl.when(s == 0)
    def _():
        pltpu.sync_copy(acc_sh, nump_hbm.at[c])


def _sc_scatter_a2a(wue, dst, zeros):
    k = pl.kernel(
        _sc_scatter_a2a_body,
        mesh=_mesh(),
        out_type=jax.ShapeDtypeStruct((NC, N_ATOM, H), jnp.float32),
        scratch_types=[
            pltpu.VMEM((128, H), jnp.float32),
            pltpu.VMEM((128,), jnp.int32),
            pltpu.VMEM((16, H), jnp.float32),
            pltpu.VMEM((16,), jnp.int32),
            pltpu.VMEM_SHARED((N_ATOM, H), jnp.float32),
            pltpu.SemaphoreType.DMA,
            pltpu.SemaphoreType.DMA,
        ],
    )
    return k(wue, dst, zeros)


def _sc_sdenom_body(n_nodes, ch, base_of, nit_of, ee_hbm, dst_hbm, sp_hbm,
                    ee_v, di_v, sacc_v):
    c = lax.axis_index("c")
    s = lax.axis_index("s")
    wid = c * NS + s

    def zs(i, _):
        sacc_v[pl.ds(i * 16, 16)] = jnp.zeros((16,), jnp.float32)
        return 0

    lax.fori_loop(0, n_nodes // 16, zs, 0)

    def body(i, _):
        base = base_of(wid, i)
        pltpu.sync_copy(ee_hbm.at[pl.ds(base, ch)], ee_v)
        pltpu.sync_copy(dst_hbm.at[pl.ds(base, ch)], di_v)
        for g in range(ch // 16):
            sl = pl.ds(g * 16, 16)
            plsc.addupdate_scatter(sacc_v, [di_v[sl]], ee_v[sl])
        return 0

    lax.fori_loop(0, nit_of(wid), body, 0)
    pltpu.sync_copy(sacc_v, sp_hbm.at[wid])


def _sc_sdenom(ee, dst, n_nodes, ch, base_of, nit_of):
    body = functools.partial(_sc_sdenom_body, n_nodes, ch, base_of, nit_of)
    k = pl.kernel(
        body,
        mesh=_mesh(),
        out_type=jax.ShapeDtypeStruct((NW, n_nodes), jnp.float32),
        scratch_types=[
            pltpu.VMEM((ch,), jnp.float32),
            pltpu.VMEM((ch,), jnp.int32),
            pltpu.VMEM((n_nodes,), jnp.float32),
        ],
        compiler_params=pltpu.CompilerParams(needs_layout_passes=False),
    )
    return k(ee, dst)


def _node_upd_body(np_ref, sp_ref, x_ref, wx_ref, wa_ref, b_ref, out_ref):
    num = np_ref[0] + np_ref[1]
    s = jnp.sum(sp_ref[...], axis=0)
    agg = num / (s[:, None] + 1e-9)
    out_ref[...] = jnp.maximum(
        jnp.dot(x_ref[...], wx_ref[...], preferred_element_type=jnp.float32)
        + jnp.dot(agg, wa_ref[...], preferred_element_type=jnp.float32)
        + b_ref[...], 0.0)


def _node_upd(nump, sp, x, wx, wa, b2d):
    n = x.shape[0]
    return pl.pallas_call(
        _node_upd_body,
        out_shape=jax.ShapeDtypeStruct((n, H), jnp.float32),
    )(nump, sp, x, wx, wa, b2d)


def _sc_a2f_body(uaf_hbm, src_hbm, dst_hbm, z_hbm, aggp_hbm,
                 si_v, di_v, rows_v, acc_sh, sem):
    c = lax.axis_index("c")
    s = lax.axis_index("s")
    wid = c * NS + s
    _zero_acc(z_hbm, acc_sh, s, N_FG)
    plsc.subcore_barrier()

    nit = jnp.where(wid < NW - 1, 4, 1)

    def body(i, _):
        base = wid * 320 + i * 80
        pltpu.sync_copy(src_hbm.at[pl.ds(base, 80)], si_v)
        pltpu.sync_copy(dst_hbm.at[pl.ds(base, 80)], di_v)
        pltpu.async_copy(uaf_hbm.at[si_v], rows_v, sem).wait()
        pltpu.sync_copy(rows_v, acc_sh.at[di_v], add=True)
        return 0

    lax.fori_loop(0, nit, body, 0)
    plsc.subcore_barrier()

    @pl.when(s == 0)
    def _():
        pltpu.sync_copy(acc_sh, aggp_hbm.at[c])


def _sc_a2f(uaf, src, dst, zeros):
    k = pl.kernel(
        _sc_a2f_body,
        mesh=_mesh(),
        out_type=jax.ShapeDtypeStruct((NC, N_FG, H), jnp.float32),
        scratch_types=[
            pltpu.VMEM((80,), jnp.int32),
            pltpu.VMEM((80,), jnp.int32),
            pltpu.VMEM((80, H), jnp.float32),
            pltpu.VMEM_SHARED((N_FG, H), jnp.float32),
            pltpu.SemaphoreType.DMA,
        ],
    )
    return k(uaf, src, dst, zeros)


def _fg_proj_body(aggp_ref, ff_ref, wf_ref, bfe_ref, uff_ref, pfs_ref, pfd_ref):
    aggm = aggp_ref[0] + aggp_ref[1]
    uff = jnp.concatenate([aggm, ff_ref[...]], axis=-1)
    uff_ref[...] = uff
    pfs_ref[...] = jnp.dot(uff, wf_ref[0:UFF, :],
                           preferred_element_type=jnp.float32)
    pfd_ref[...] = (jnp.dot(uff, wf_ref[UFF:2 * UFF, :],
                            preferred_element_type=jnp.float32)
                    + bfe_ref[...])


def _fg_proj(aggp, ff, wf_e, bfe2d):
    return pl.pallas_call(
        _fg_proj_body,
        out_shape=[
            jax.ShapeDtypeStruct((N_FG, UFF), jnp.float32),
            jax.ShapeDtypeStruct((N_FG, H), jnp.float32),
            jax.ShapeDtypeStruct((N_FG, H), jnp.float32),
        ],
    )(aggp, ff, wf_e, bfe2d)


def _sc_f2f_gather_body(ps_hbm, pd_hbm, src_hbm, dst_hbm, g_hbm,
                        si_v, di_v, gs_v, gd_v, sem1, sem2):
    c = lax.axis_index("c")
    s = lax.axis_index("s")
    wid = c * NS + s
    nit = jnp.where(wid < NW - 1, 8, 2)

    def body(i, _):
        base = wid * 512 + i * 64
        pltpu.sync_copy(src_hbm.at[pl.ds(base, 64)], si_v)
        pltpu.sync_copy(dst_hbm.at[pl.ds(base, 64)], di_v)
        cp1 = pltpu.async_copy(ps_hbm.at[si_v], gs_v, sem1)
        cp2 = pltpu.async_copy(pd_hbm.at[di_v], gd_v, sem2)
        cp1.wait()
        cp2.wait()

        def add_row(r, _):
            for cg in range(H // 16):
                sl = pl.ds(cg * 16, 16)
                gs_v[r, sl] = gs_v[r, sl] + gd_v[r, sl]
            return 0

        lax.fori_loop(0, 64, add_row, 0)
        pltpu.sync_copy(gs_v, g_hbm.at[pl.ds(base, 64)])
        return 0

    lax.fori_loop(0, nit, body, 0)


def _sc_f2f_gather(pfs, pfd, src, dst):
    k = pl.kernel(
        _sc_f2f_gather_body,
        mesh=_mesh(),
        out_type=jax.ShapeDtypeStruct((E_F2F, H), jnp.float32),
        scratch_types=[
            pltpu.VMEM((64,), jnp.int32),
            pltpu.VMEM((64,), jnp.int32),
            pltpu.VMEM((64, H), jnp.float32),
            pltpu.VMEM((64, H), jnp.float32),
            pltpu.SemaphoreType.DMA,
            pltpu.SemaphoreType.DMA,
        ],
    )
    return k(pfs, pfd, src, dst)


def _f2f_act_body(g_ref, wa_ref, ba_ref, ue_ref, ee_ref, wue_ref):
    ue = jnp.maximum(g_ref[...], 0.0)
    ue_ref[...] = ue
    logit = jnp.dot(ue, wa_ref[...], preferred_element_type=jnp.float32)
    ee = jnp.exp(logit + ba_ref[...])
    ee_ref[...] = ee
    wue_ref[...] = ue * ee


def _f2f_act(g, wa, ba2d, blk):
    n = g.shape[0]
    return pl.pallas_call(
        _f2f_act_body,
        grid=(n // blk,),
        in_specs=[
            pl.BlockSpec((blk, H), lambda i: (i, 0)),
            pl.BlockSpec(wa.shape, lambda i: (0, 0)),
            pl.BlockSpec(ba2d.shape, lambda i: (0, 0)),
        ],
        out_specs=[
            pl.BlockSpec((blk, H), lambda i: (i, 0)),
            pl.BlockSpec((blk, 1), lambda i: (i, 0)),
            pl.BlockSpec((blk, H), lambda i: (i, 0)),
        ],
        out_shape=[
            jax.ShapeDtypeStruct((n, H), jnp.float32),
            jax.ShapeDtypeStruct((n, 1), jnp.float32),
            jax.ShapeDtypeStruct((n, H), jnp.float32),
        ],
    )(g, wa, ba2d)


def _fg_upd_body(np_ref, sp_ref, uff_ref, wf_ref, b_ref, out_ref):
    num = np_ref[0] + np_ref[1]
    s = jnp.sum(sp_ref[...], axis=0)
    agg = num / (s[:, None] + 1e-9)
    uff = uff_ref[...]
    out_ref[...] = jnp.maximum(
        jnp.dot(uff, wf_ref[0:UFF, :], preferred_element_type=jnp.float32)
        + jnp.dot(agg, wf_ref[UFF:UFF + H, :],
                  preferred_element_type=jnp.float32)
        + b_ref[...], 0.0)


def _fg_upd(nump, sp, uff, wf_n, b2d):
    return pl.pallas_call(
        _fg_upd_body,
        out_shape=jax.ShapeDtypeStruct((N_FG, H), jnp.float32),
    )(nump, sp, uff, wf_n, b2d)


def kernel(af, bf, ff, W_e, b_e, W_a, b_a, W_n, b_n, Wf_e, bf_e, Wf_a, bf_a,
           Wf_n, bf_n, a2a_edge_index, a2f_src, a2f_dst, f2f_edge_index):
    src = a2a_edge_index[0].astype(jnp.int32)
    dst = a2a_edge_index[1].astype(jnp.int32)
    fsrc = f2f_edge_index[0].astype(jnp.int32)
    fdst = f2f_edge_index[1].astype(jnp.int32)
    asrc = a2f_src.astype(jnp.int32)
    adst = a2f_dst.astype(jnp.int32)

    ps, pd = _proj_node(af, W_e[:D], W_e[D:2 * D], b_e.reshape(1, H),
                        N_ATOM, 1000)
    g = _sc_gather_add(ps, pd, src, dst, E_A2A, 80, 5)
    ubf, ee, wue = _edge_act(g, bf, W_e[2 * D:], W_a, b_a.reshape(1, 1), 2000)
    ee = ee.reshape(E_A2A)
    zeros = jnp.zeros((200, H), jnp.float32)
    epw = E_A2A // NW
    nump = _sc_scatter_a2a(wue, dst, zeros)
    sp = _sc_sdenom(ee, dst, N_ATOM, 2000,
                    lambda wid, i: wid * epw + i * 2000,
                    lambda wid: epw // 2000)
    uaf = _node_upd(nump, sp, af, W_n[:D], W_n[D:], b_n.reshape(1, H))
    aggp = _sc_a2f(uaf, asrc, adst, zeros)
    uff, pfs, pfd = _fg_proj(aggp, ff, Wf_e, bf_e.reshape(1, H))
    gf = _sc_f2f_gather(pfs, pfd, fsrc, fdst)
    uef, eef, wuef = _f2f_act(gf, Wf_a, bf_a.reshape(1, 1), 2000)
    eef = eef.reshape(E_F2F)
    numfp = _sc_scatter(wuef, fdst, zeros, N_FG, 64,
                        lambda wid, i: wid * 512 + i * 64,
                        lambda wid: jnp.where(wid < NW - 1, 8, 2))
    sfp = _sc_sdenom(eef, fdst, N_FG, 64,
                     lambda wid, i: wid * 512 + i * 64,
                     lambda wid: jnp.where(wid < NW - 1, 8, 2))
    conv_uff = _fg_upd(numfp, sfp, uff, Wf_n, bf_n.reshape(1, H))
    return (uaf, ubf, conv_uff)
